# Initial kernel scaffold; baseline (speedup 1.0000x reference)
import jax
import jax.numpy as jnp
from jax.experimental import pallas as pl


def kernel(edge_index, user_embed, laplacian_lambda_max, cheb_W, cheb_b, gat_Wsrc, gat_bsrc, gat_Wdst, gat_bdst, gat_attn):
    raise NotImplementedError("write your pallas kernel here")


# speedup vs baseline: 5.3113x; 5.3113x over previous
import functools

import jax
import jax.numpy as jnp
from jax import lax
from jax.experimental import pallas as pl
from jax.experimental.pallas import tpu as pltpu
from jax.experimental.pallas import tpu_sc as plsc

N = 10000
E = 320000
D = 128

NC = 2
NS = 16
NW = NC * NS
L = 16

EPT = E // NW
CH = 80
NCHUNK = EPT // CH
RPT = N // NS

_mesh = plsc.VectorSubcoreMesh(core_axis_name="c", subcore_axis_name="s",
                               num_cores=NC, num_subcores=NS)
_sc_params = pltpu.CompilerParams(use_tc_tiling_on_sc=False,
                                  needs_layout_passes=False)


def _tile_base(unused=None):
    core = lax.axis_index("c")
    sub = lax.axis_index("s")
    return core, sub, (core * NS + sub) * EPT


def _sc_deg_body(dst_hbm, z16_hbm, out_hbm, acc_sh, idx_d, ones_v, sem):
    core, sub, base0 = _tile_base()

    pltpu.sync_copy(z16_hbm.at[pl.ds(sub * RPT, RPT)],
                    acc_sh.at[pl.ds(sub * RPT, RPT)])

    @pl.loop(0, CH)
    def _ones(i):
        ones_v[i, :] = jnp.ones((L,), jnp.float32)

    plsc.subcore_barrier()

    @pl.loop(0, NCHUNK)
    def _chunk(ci):
        base = pl.multiple_of(base0 + ci * CH, 8)
        pltpu.async_copy(dst_hbm.at[pl.ds(base, CH)], idx_d, sem).wait()
        pltpu.sync_copy(ones_v, acc_sh.at[idx_d], add=True)

    plsc.subcore_barrier()
    pltpu.sync_copy(acc_sh.at[pl.ds(sub * RPT, RPT)],
                    out_hbm.at[core, pl.ds(sub * RPT, RPT)])


@jax.jit
def _sc_deg(dst, z16):
    k = pl.kernel(
        _sc_deg_body,
        out_type=jax.ShapeDtypeStruct((NC, N, L), jnp.float32),
        mesh=_mesh,
        compiler_params=_sc_params,
        scratch_types=[
            pltpu.VMEM_SHARED((N, L), jnp.float32),
            pltpu.VMEM((CH,), jnp.int32),
            pltpu.VMEM((CH, L), jnp.float32),
            pltpu.SemaphoreType.DMA,
        ],
    )
    return k(dst, z16)


def _sc_segsum_body(src_hbm, dst_hbm, table_hbm, zd_hbm, out_hbm,
                    acc_sh, idx_s, idx_d, rows, sem):
    core, sub, base0 = _tile_base()

    pltpu.sync_copy(zd_hbm.at[pl.ds(sub * RPT, RPT)],
                    acc_sh.at[pl.ds(sub * RPT, RPT)])
    plsc.subcore_barrier()

    @pl.loop(0, NCHUNK)
    def _chunk(ci):
        base = pl.multiple_of(base0 + ci * CH, 8)
        pltpu.async_copy(src_hbm.at[pl.ds(base, CH)], idx_s, sem).wait()
        pltpu.async_copy(dst_hbm.at[pl.ds(base, CH)], idx_d, sem).wait()
        pltpu.async_copy(table_hbm.at[idx_s], rows, sem).wait()
        pltpu.sync_copy(rows, acc_sh.at[idx_d], add=True)

    plsc.subcore_barrier()
    for k in range(RPT // 125):
        r0 = sub * RPT + k * 125
        pltpu.sync_copy(acc_sh.at[pl.ds(r0, 125)], out_hbm.at[core, pl.ds(r0, 125)])


@jax.jit
def _sc_segsum(src, dst, table, zd):
    k = pl.kernel(
        _sc_segsum_body,
        out_type=jax.ShapeDtypeStruct((NC, N, D), jnp.float32),
        mesh=_mesh,
        compiler_params=_sc_params,
        scratch_types=[
            pltpu.VMEM_SHARED((N, D), jnp.float32),
            pltpu.VMEM((CH,), jnp.int32),
            pltpu.VMEM((CH,), jnp.int32),
            pltpu.VMEM((CH, D), jnp.float32),
            pltpu.SemaphoreType.DMA,
        ],
    )
    return k(src, dst, table, zd)


def _sc_gat_body(src_hbm, dst_hbm, fs_hbm, fd_hbm, attn_hbm, zd_hbm, z16_hbm,
                 out_hbm, den_hbm,
                 acc_sh, den_sh, idx_s, idx_d, arows, brows, mrows, drows,
                 attn_v, pmat, eebuf, sem):
    core, sub, base0 = _tile_base()

    pltpu.sync_copy(zd_hbm.at[pl.ds(sub * RPT, RPT)],
                    acc_sh.at[pl.ds(sub * RPT, RPT)])
    pltpu.sync_copy(z16_hbm.at[pl.ds(sub * RPT, RPT)],
                    den_sh.at[pl.ds(sub * RPT, RPT)])
    pltpu.sync_copy(attn_hbm, attn_v)
    plsc.subcore_barrier()

    lanes = lax.iota(jnp.int32, L)

    def lane_bcast(vec, i):
        idx = jnp.full((L, 1), i, jnp.int32)
        dnums = lax.GatherDimensionNumbers(
            offset_dims=(), collapsed_slice_dims=(0,), start_index_map=(0,))
        return lax.gather(vec, idx, dnums, (1,),
                          mode=lax.GatherScatterMode.PROMISE_IN_BOUNDS)

    @pl.loop(0, NCHUNK)
    def _chunk(ci):
        base = pl.multiple_of(base0 + ci * CH, 8)
        pltpu.async_copy(src_hbm.at[pl.ds(base, CH)], idx_s, sem).wait()
        pltpu.async_copy(dst_hbm.at[pl.ds(base, CH)], idx_d, sem).wait()
        pltpu.async_copy(fs_hbm.at[idx_s], arows, sem).wait()
        pltpu.async_copy(fd_hbm.at[idx_d], brows, sem).wait()

        @pl.loop(0, CH, step=L)
        def _grp(g):
            for i in range(L):
                acc = jnp.zeros((L,), jnp.float32)
                for j in range(D // L):
                    a = arows[g + i, pl.ds(j * L, L)]
                    b = brows[g + i, pl.ds(j * L, L)]
                    s = a + b
                    s = jnp.where(s >= 0.0, s, s * jnp.float32(0.2))
                    acc = acc + s * attn_v[pl.ds(j * L, L)]
                pmat[i, :] = acc
            e16 = jnp.zeros((L,), jnp.float32)
            for kk in range(L):
                col = plsc.load_gather(pmat, [lanes, jnp.full((L,), kk, jnp.int32)])
                e16 = e16 + col
            ee16 = jnp.exp(e16)
            eebuf[...] = ee16
            for i in range(L):
                bc = lane_bcast(ee16, i)
                for j in range(D // L):
                    mrows[g + i, pl.ds(j * L, L)] = (
                        arows[g + i, pl.ds(j * L, L)] * bc)
                drows[g + i, :] = bc

        pltpu.sync_copy(mrows, acc_sh.at[idx_d], add=True)
        pltpu.sync_copy(drows, den_sh.at[idx_d], add=True)

    plsc.subcore_barrier()
    for k in range(RPT // 125):
        r0 = sub * RPT + k * 125
        pltpu.sync_copy(acc_sh.at[pl.ds(r0, 125)], out_hbm.at[core, pl.ds(r0, 125)])
    pltpu.sync_copy(den_sh.at[pl.ds(sub * RPT, RPT)],
                    den_hbm.at[core, pl.ds(sub * RPT, RPT)])


@jax.jit
def _sc_gat(src, dst, fs, fd, attn, zd, z16):
    k = pl.kernel(
        _sc_gat_body,
        out_type=(jax.ShapeDtypeStruct((NC, N, D), jnp.float32),
                  jax.ShapeDtypeStruct((NC, N, L), jnp.float32)),
        mesh=_mesh,
        compiler_params=_sc_params,
        scratch_types=[
            pltpu.VMEM_SHARED((N, D), jnp.float32),
            pltpu.VMEM_SHARED((N, L), jnp.float32),
            pltpu.VMEM((CH,), jnp.int32),
            pltpu.VMEM((CH,), jnp.int32),
            pltpu.VMEM((CH, D), jnp.float32),
            pltpu.VMEM((CH, D), jnp.float32),
            pltpu.VMEM((CH, D), jnp.float32),
            pltpu.VMEM((CH, L), jnp.float32),
            pltpu.VMEM((D,), jnp.float32),
            pltpu.VMEM((L, L), jnp.float32),
            pltpu.VMEM((L,), jnp.float32),
            pltpu.SemaphoreType.DMA,
        ],
    )
    return k(src, dst, fs, fd, attn, zd, z16)


BLK = 1000
GRID = N // BLK


def _row_spec(width=D):
    return pl.BlockSpec((BLK, width), lambda i: (i, 0))


def _full_spec(shape):
    ndim = len(shape)
    return pl.BlockSpec(shape, lambda i: (0,) * ndim)


def _tc_deg_h_body(d0_ref, d1_ref, x_ref, dinv_ref, h_ref):
    deg = d0_ref[:, 0:1] + d1_ref[:, 0:1]
    dinv = lax.rsqrt(jnp.maximum(deg, 1.0))
    dinv_ref[...] = jnp.broadcast_to(dinv, (BLK, L))
    h_ref[...] = x_ref[...] * dinv


@jax.jit
def _tc_deg_h(degp, x0):
    return pl.pallas_call(
        _tc_deg_h_body,
        grid=(GRID,),
        in_specs=[_row_spec(L), _row_spec(L), _row_spec()],
        out_specs=[_row_spec(L), _row_spec()],
        out_shape=[jax.ShapeDtypeStruct((N, L), jnp.float32),
                   jax.ShapeDtypeStruct((N, D), jnp.float32)],
    )(degp[0], degp[1], x0)


def _tc_step1_body(s0_ref, s1_ref, x0_ref, dinv_ref, rn_ref, x1_ref, h1_ref):
    rn = rn_ref[0]
    dv = dinv_ref[:, 0:1]
    agg = (s0_ref[...] + s1_ref[...]) * dv
    x1 = agg * (-rn) + x0_ref[...] * (rn - 1.0)
    x1_ref[...] = x1
    h1_ref[...] = x1 * dv


@jax.jit
def _tc_step1(s0, s1, x0, dinv, rn):
    return pl.pallas_call(
        _tc_step1_body,
        grid=(GRID,),
        in_specs=[_row_spec(), _row_spec(), _row_spec(), _row_spec(L),
                  pl.BlockSpec(memory_space=pltpu.SMEM)],
        out_specs=[_row_spec(), _row_spec()],
        out_shape=[jax.ShapeDtypeStruct((N, D), jnp.float32),
                   jax.ShapeDtypeStruct((N, D), jnp.float32)],
    )(s0, s1, x0, dinv, rn)


def _tc_step2_body(s0_ref, s1_ref, x0_ref, x1_ref, dinv_ref, rn_ref,
                   w0_ref, w1_ref, w2_ref, b_ref, x_ref, h_ref):
    rn = rn_ref[0]
    dv = dinv_ref[:, 0:1]
    agg = (s0_ref[...] + s1_ref[...]) * dv
    x2 = agg * (-2.0 * rn) + x1_ref[...] * (2.0 * (rn - 1.0)) - x0_ref[...]
    acc = jnp.dot(x0_ref[...], w0_ref[...],
                  preferred_element_type=jnp.float32,
                  precision=lax.Precision.HIGHEST)
    acc += jnp.dot(x1_ref[...], w1_ref[...],
                   preferred_element_type=jnp.float32,
                  precision=lax.Precision.HIGHEST)
    acc += jnp.dot(x2, w2_ref[...], preferred_element_type=jnp.float32,
                  precision=lax.Precision.HIGHEST)
    xn = jnp.maximum(acc + b_ref[...], 0.0)
    x_ref[...] = xn
    h_ref[...] = xn * dv


@jax.jit
def _tc_step2(s0, s1, x0, x1, dinv, rn, w0, w1, w2, b):
    return pl.pallas_call(
        _tc_step2_body,
        grid=(GRID,),
        in_specs=[_row_spec(), _row_spec(), _row_spec(), _row_spec(),
                  _row_spec(L),
                  pl.BlockSpec(memory_space=pltpu.SMEM),
                  _full_spec((D, D)), _full_spec((D, D)), _full_spec((D, D)),
                  _full_spec((1, D))],
        out_specs=[_row_spec(), _row_spec()],
        out_shape=[jax.ShapeDtypeStruct((N, D), jnp.float32),
                   jax.ShapeDtypeStruct((N, D), jnp.float32)],
    )(s0, s1, x0, x1, dinv, rn, w0, w1, w2, b)


def _tc_fsfd_body(x_ref, ws_ref, bs_ref, wd_ref, bd_ref, fs_ref, fd_ref):
    x = x_ref[...]
    fs_ref[...] = jnp.dot(x, ws_ref[...],
                          preferred_element_type=jnp.float32,
                  precision=lax.Precision.HIGHEST) + bs_ref[...]
    fd_ref[...] = jnp.dot(x, wd_ref[...],
                          preferred_element_type=jnp.float32,
                  precision=lax.Precision.HIGHEST) + bd_ref[...]


@jax.jit
def _tc_fsfd(x, ws, bs, wd, bd):
    return pl.pallas_call(
        _tc_fsfd_body,
        grid=(GRID,),
        in_specs=[_row_spec(), _full_spec((D, D)), _full_spec((1, D)),
                  _full_spec((D, D)), _full_spec((1, D))],
        out_specs=[_row_spec(), _row_spec()],
        out_shape=[jax.ShapeDtypeStruct((N, D), jnp.float32),
                   jax.ShapeDtypeStruct((N, D), jnp.float32)],
    )(x, ws, bs, wd, bd)


def _tc_final_body(o0_ref, o1_ref, d0_ref, d1_ref, out_ref):
    den = d0_ref[:, 0:1] + d1_ref[:, 0:1]
    num = o0_ref[...] + o1_ref[...]
    out_ref[...] = jnp.where(den > 0.0, num / jnp.maximum(den, 1e-38), 0.0)


@jax.jit
def _tc_final(o0, o1, d0, d1):
    return pl.pallas_call(
        _tc_final_body,
        grid=(GRID,),
        in_specs=[_row_spec(), _row_spec(), _row_spec(L), _row_spec(L)],
        out_specs=_row_spec(),
        out_shape=jax.ShapeDtypeStruct((N, D), jnp.float32),
    )(o0, o1, d0, d1)


def kernel(edge_index, user_embed, laplacian_lambda_max, cheb_W, cheb_b,
           gat_Wsrc, gat_bsrc, gat_Wdst, gat_bdst, gat_attn):
    src = edge_index[0]
    dst = edge_index[1]
    rn = (2.0 / laplacian_lambda_max).astype(jnp.float32)
    w0 = cheb_W[:D]
    w1 = cheb_W[D:2 * D]
    w2 = cheb_W[2 * D:]
    bias = cheb_b.reshape(1, D)
    bs = gat_bsrc.reshape(1, D)
    bd = gat_bdst.reshape(1, D)
    attn = gat_attn.reshape(D)

    zd = jnp.zeros((N, D), jnp.float32)
    z16 = jnp.zeros((N, L), jnp.float32)

    degp = _sc_deg(dst, z16)
    dinv, h = _tc_deg_h(degp, user_embed)

    def cheb(x0, h0):
        s = _sc_segsum(src, dst, h0, zd)
        x1, h1 = _tc_step1(s[0], s[1], x0, dinv, rn)
        s2 = _sc_segsum(src, dst, h1, zd)
        return _tc_step2(s2[0], s2[1], x0, x1, dinv, rn, w0, w1, w2, bias)

    x1, h1 = cheb(user_embed, h)
    x2, _ = cheb(x1, h1)

    fs, fd = _tc_fsfd(x2, gat_Wsrc, bs, gat_Wdst, bd)
    acc, den = _sc_gat(src, dst, fs, fd, attn, zd, z16)
    return _tc_final(acc[0], acc[1], den[0], den[1])


# speedup vs baseline: 7.2977x; 1.3740x over previous
import functools

import jax
import jax.numpy as jnp
from jax import lax
from jax.experimental import pallas as pl
from jax.experimental.pallas import tpu as pltpu
from jax.experimental.pallas import tpu_sc as plsc

N = 10000
E = 320000
D = 128

NC = 2
NS = 16
NW = NC * NS
L = 16

EPT = E // NW
CH = 80
NCHUNK = EPT // CH
RPT = N // NS

_mesh = plsc.VectorSubcoreMesh(core_axis_name="c", subcore_axis_name="s",
                               num_cores=NC, num_subcores=NS)
_sc_params = pltpu.CompilerParams(use_tc_tiling_on_sc=False,
                                  needs_layout_passes=False)


def _tile_base(unused=None):
    core = lax.axis_index("c")
    sub = lax.axis_index("s")
    return core, sub, (core * NS + sub) * EPT


def _sc_deg_body(dst_hbm, z16_hbm, out_hbm, acc_sh, idx_d, ones_v, sem):
    core, sub, base0 = _tile_base()

    pltpu.sync_copy(z16_hbm.at[pl.ds(sub * RPT, RPT)],
                    acc_sh.at[pl.ds(sub * RPT, RPT)])

    @pl.loop(0, CH)
    def _ones(i):
        ones_v[i, :] = jnp.ones((L,), jnp.float32)

    plsc.subcore_barrier()

    @pl.loop(0, NCHUNK)
    def _chunk(ci):
        base = pl.multiple_of(base0 + ci * CH, 8)
        pltpu.async_copy(dst_hbm.at[pl.ds(base, CH)], idx_d, sem).wait()
        pltpu.sync_copy(ones_v, acc_sh.at[idx_d], add=True)

    plsc.subcore_barrier()
    pltpu.sync_copy(acc_sh.at[pl.ds(sub * RPT, RPT)],
                    out_hbm.at[core, pl.ds(sub * RPT, RPT)])


@jax.jit
def _sc_deg(dst, z16):
    k = pl.kernel(
        _sc_deg_body,
        out_type=jax.ShapeDtypeStruct((NC, N, L), jnp.float32),
        mesh=_mesh,
        compiler_params=_sc_params,
        scratch_types=[
            pltpu.VMEM_SHARED((N, L), jnp.float32),
            pltpu.VMEM((CH,), jnp.int32),
            pltpu.VMEM((CH, L), jnp.float32),
            pltpu.SemaphoreType.DMA,
        ],
    )
    return k(dst, z16)


def _sc_segsum_body(src3_hbm, dst3_hbm, table_hbm, zd_hbm, out_hbm,
                    acc_sh, idxs, idxd, rows0, rows1, semi, sem0, sem1):
    core, sub, base0 = _tile_base()
    wid = core * NS + sub

    cs = pltpu.async_copy(src3_hbm.at[wid], idxs, semi)
    cd = pltpu.async_copy(dst3_hbm.at[wid], idxd, semi)
    pltpu.sync_copy(zd_hbm.at[pl.ds(sub * RPT, RPT)],
                    acc_sh.at[pl.ds(sub * RPT, RPT)])
    cs.wait()
    cd.wait()
    plsc.subcore_barrier()

    pltpu.async_copy(table_hbm.at[idxs.at[0]], rows0, sem0)

    @pl.loop(0, NCHUNK - 1, step=2)
    def _chunk(ci0):
        pltpu.make_async_copy(table_hbm.at[idxs.at[ci0]], rows0, sem0).wait()
        pltpu.async_copy(table_hbm.at[idxs.at[ci0 + 1]], rows1, sem1)
        pltpu.sync_copy(rows0, acc_sh.at[idxd.at[ci0]], add=True)
        pltpu.make_async_copy(table_hbm.at[idxs.at[ci0 + 1]], rows1, sem1).wait()
        pltpu.async_copy(table_hbm.at[idxs.at[ci0 + 2]], rows0, sem0)
        pltpu.sync_copy(rows1, acc_sh.at[idxd.at[ci0 + 1]], add=True)

    last = NCHUNK - 1
    pltpu.make_async_copy(table_hbm.at[idxs.at[last]], rows0, sem0).wait()
    pltpu.sync_copy(rows0, acc_sh.at[idxd.at[last]], add=True)

    plsc.subcore_barrier()
    for k in range(RPT // 125):
        r0 = sub * RPT + k * 125
        pltpu.sync_copy(acc_sh.at[pl.ds(r0, 125)], out_hbm.at[core, pl.ds(r0, 125)])


@jax.jit
def _sc_segsum(src3, dst3, table, zd):
    k = pl.kernel(
        _sc_segsum_body,
        out_type=jax.ShapeDtypeStruct((NC, N, D), jnp.float32),
        mesh=_mesh,
        compiler_params=_sc_params,
        scratch_types=[
            pltpu.VMEM_SHARED((N, D), jnp.float32),
            pltpu.VMEM((NCHUNK, CH), jnp.int32),
            pltpu.VMEM((NCHUNK, CH), jnp.int32),
            pltpu.VMEM((CH, D), jnp.float32),
            pltpu.VMEM((CH, D), jnp.float32),
            pltpu.SemaphoreType.DMA,
            pltpu.SemaphoreType.DMA,
            pltpu.SemaphoreType.DMA,
        ],
    )
    return k(src3, dst3, table, zd)


def _sc_gat_body(src_hbm, dst_hbm, fs_hbm, fd_hbm, attn_hbm, zd_hbm, z16_hbm,
                 out_hbm, den_hbm,
                 acc_sh, den_sh, idx_s, idx_d, arows, brows, mrows, drows,
                 attn_v, pmat, eebuf, sem):
    core, sub, base0 = _tile_base()

    pltpu.sync_copy(zd_hbm.at[pl.ds(sub * RPT, RPT)],
                    acc_sh.at[pl.ds(sub * RPT, RPT)])
    pltpu.sync_copy(z16_hbm.at[pl.ds(sub * RPT, RPT)],
                    den_sh.at[pl.ds(sub * RPT, RPT)])
    pltpu.sync_copy(attn_hbm, attn_v)
    plsc.subcore_barrier()

    lanes = lax.iota(jnp.int32, L)

    def lane_bcast(vec, i):
        idx = jnp.full((L, 1), i, jnp.int32)
        dnums = lax.GatherDimensionNumbers(
            offset_dims=(), collapsed_slice_dims=(0,), start_index_map=(0,))
        return lax.gather(vec, idx, dnums, (1,),
                          mode=lax.GatherScatterMode.PROMISE_IN_BOUNDS)

    @pl.loop(0, NCHUNK)
    def _chunk(ci):
        base = pl.multiple_of(base0 + ci * CH, 8)
        pltpu.async_copy(src_hbm.at[pl.ds(base, CH)], idx_s, sem).wait()
        pltpu.async_copy(dst_hbm.at[pl.ds(base, CH)], idx_d, sem).wait()
        pltpu.async_copy(fs_hbm.at[idx_s], arows, sem).wait()
        pltpu.async_copy(fd_hbm.at[idx_d], brows, sem).wait()

        @pl.loop(0, CH, step=L)
        def _grp(g):
            for i in range(L):
                acc = jnp.zeros((L,), jnp.float32)
                for j in range(D // L):
                    a = arows[g + i, pl.ds(j * L, L)]
                    b = brows[g + i, pl.ds(j * L, L)]
                    s = a + b
                    s = jnp.where(s >= 0.0, s, s * jnp.float32(0.2))
                    acc = acc + s * attn_v[pl.ds(j * L, L)]
                pmat[i, :] = acc
            e16 = jnp.zeros((L,), jnp.float32)
            for kk in range(L):
                col = plsc.load_gather(pmat, [lanes, jnp.full((L,), kk, jnp.int32)])
                e16 = e16 + col
            ee16 = jnp.exp(e16)
            eebuf[...] = ee16
            for i in range(L):
                bc = lane_bcast(ee16, i)
                for j in range(D // L):
                    mrows[g + i, pl.ds(j * L, L)] = (
                        arows[g + i, pl.ds(j * L, L)] * bc)
                drows[g + i, :] = bc

        pltpu.sync_copy(mrows, acc_sh.at[idx_d], add=True)
        pltpu.sync_copy(drows, den_sh.at[idx_d], add=True)

    plsc.subcore_barrier()
    for k in range(RPT // 125):
        r0 = sub * RPT + k * 125
        pltpu.sync_copy(acc_sh.at[pl.ds(r0, 125)], out_hbm.at[core, pl.ds(r0, 125)])
    pltpu.sync_copy(den_sh.at[pl.ds(sub * RPT, RPT)],
                    den_hbm.at[core, pl.ds(sub * RPT, RPT)])


@jax.jit
def _sc_gat(src, dst, fs, fd, attn, zd, z16):
    k = pl.kernel(
        _sc_gat_body,
        out_type=(jax.ShapeDtypeStruct((NC, N, D), jnp.float32),
                  jax.ShapeDtypeStruct((NC, N, L), jnp.float32)),
        mesh=_mesh,
        compiler_params=_sc_params,
        scratch_types=[
            pltpu.VMEM_SHARED((N, D), jnp.float32),
            pltpu.VMEM_SHARED((N, L), jnp.float32),
            pltpu.VMEM((CH,), jnp.int32),
            pltpu.VMEM((CH,), jnp.int32),
            pltpu.VMEM((CH, D), jnp.float32),
            pltpu.VMEM((CH, D), jnp.float32),
            pltpu.VMEM((CH, D), jnp.float32),
            pltpu.VMEM((CH, L), jnp.float32),
            pltpu.VMEM((D,), jnp.float32),
            pltpu.VMEM((L, L), jnp.float32),
            pltpu.VMEM((L,), jnp.float32),
            pltpu.SemaphoreType.DMA,
        ],
    )
    return k(src, dst, fs, fd, attn, zd, z16)


BLK = 1000
GRID = N // BLK


def _row_spec(width=D):
    return pl.BlockSpec((BLK, width), lambda i: (i, 0))


def _full_spec(shape):
    ndim = len(shape)
    return pl.BlockSpec(shape, lambda i: (0,) * ndim)


def _tc_deg_h_body(d0_ref, d1_ref, x_ref, dinv_ref, h_ref):
    deg = d0_ref[:, 0:1] + d1_ref[:, 0:1]
    dinv = lax.rsqrt(jnp.maximum(deg, 1.0))
    dinv_ref[...] = jnp.broadcast_to(dinv, (BLK, L))
    h_ref[...] = x_ref[...] * dinv


@jax.jit
def _tc_deg_h(degp, x0):
    return pl.pallas_call(
        _tc_deg_h_body,
        grid=(GRID,),
        in_specs=[_row_spec(L), _row_spec(L), _row_spec()],
        out_specs=[_row_spec(L), _row_spec()],
        out_shape=[jax.ShapeDtypeStruct((N, L), jnp.float32),
                   jax.ShapeDtypeStruct((N, D), jnp.float32)],
    )(degp[0], degp[1], x0)


def _tc_step1_body(s0_ref, s1_ref, x0_ref, dinv_ref, rn_ref, x1_ref, h1_ref):
    rn = rn_ref[0]
    dv = dinv_ref[:, 0:1]
    agg = (s0_ref[...] + s1_ref[...]) * dv
    x1 = agg * (-rn) + x0_ref[...] * (rn - 1.0)
    x1_ref[...] = x1
    h1_ref[...] = x1 * dv


@jax.jit
def _tc_step1(s0, s1, x0, dinv, rn):
    return pl.pallas_call(
        _tc_step1_body,
        grid=(GRID,),
        in_specs=[_row_spec(), _row_spec(), _row_spec(), _row_spec(L),
                  pl.BlockSpec(memory_space=pltpu.SMEM)],
        out_specs=[_row_spec(), _row_spec()],
        out_shape=[jax.ShapeDtypeStruct((N, D), jnp.float32),
                   jax.ShapeDtypeStruct((N, D), jnp.float32)],
    )(s0, s1, x0, dinv, rn)


def _tc_step2_body(s0_ref, s1_ref, x0_ref, x1_ref, dinv_ref, rn_ref,
                   w0_ref, w1_ref, w2_ref, b_ref, x_ref, h_ref):
    rn = rn_ref[0]
    dv = dinv_ref[:, 0:1]
    agg = (s0_ref[...] + s1_ref[...]) * dv
    x2 = agg * (-2.0 * rn) + x1_ref[...] * (2.0 * (rn - 1.0)) - x0_ref[...]
    acc = jnp.dot(x0_ref[...], w0_ref[...],
                  preferred_element_type=jnp.float32,
                  precision=lax.Precision.HIGHEST)
    acc += jnp.dot(x1_ref[...], w1_ref[...],
                   preferred_element_type=jnp.float32,
                  precision=lax.Precision.HIGHEST)
    acc += jnp.dot(x2, w2_ref[...], preferred_element_type=jnp.float32,
                  precision=lax.Precision.HIGHEST)
    xn = jnp.maximum(acc + b_ref[...], 0.0)
    x_ref[...] = xn
    h_ref[...] = xn * dv


@jax.jit
def _tc_step2(s0, s1, x0, x1, dinv, rn, w0, w1, w2, b):
    return pl.pallas_call(
        _tc_step2_body,
        grid=(GRID,),
        in_specs=[_row_spec(), _row_spec(), _row_spec(), _row_spec(),
                  _row_spec(L),
                  pl.BlockSpec(memory_space=pltpu.SMEM),
                  _full_spec((D, D)), _full_spec((D, D)), _full_spec((D, D)),
                  _full_spec((1, D))],
        out_specs=[_row_spec(), _row_spec()],
        out_shape=[jax.ShapeDtypeStruct((N, D), jnp.float32),
                   jax.ShapeDtypeStruct((N, D), jnp.float32)],
    )(s0, s1, x0, x1, dinv, rn, w0, w1, w2, b)


def _tc_fsfd_body(x_ref, ws_ref, bs_ref, wd_ref, bd_ref, fs_ref, fd_ref):
    x = x_ref[...]
    fs_ref[...] = jnp.dot(x, ws_ref[...],
                          preferred_element_type=jnp.float32,
                  precision=lax.Precision.HIGHEST) + bs_ref[...]
    fd_ref[...] = jnp.dot(x, wd_ref[...],
                          preferred_element_type=jnp.float32,
                  precision=lax.Precision.HIGHEST) + bd_ref[...]


@jax.jit
def _tc_fsfd(x, ws, bs, wd, bd):
    return pl.pallas_call(
        _tc_fsfd_body,
        grid=(GRID,),
        in_specs=[_row_spec(), _full_spec((D, D)), _full_spec((1, D)),
                  _full_spec((D, D)), _full_spec((1, D))],
        out_specs=[_row_spec(), _row_spec()],
        out_shape=[jax.ShapeDtypeStruct((N, D), jnp.float32),
                   jax.ShapeDtypeStruct((N, D), jnp.float32)],
    )(x, ws, bs, wd, bd)


def _tc_final_body(o0_ref, o1_ref, d0_ref, d1_ref, out_ref):
    den = d0_ref[:, 0:1] + d1_ref[:, 0:1]
    num = o0_ref[...] + o1_ref[...]
    out_ref[...] = jnp.where(den > 0.0, num / jnp.maximum(den, 1e-38), 0.0)


@jax.jit
def _tc_final(o0, o1, d0, d1):
    return pl.pallas_call(
        _tc_final_body,
        grid=(GRID,),
        in_specs=[_row_spec(), _row_spec(), _row_spec(L), _row_spec(L)],
        out_specs=_row_spec(),
        out_shape=jax.ShapeDtypeStruct((N, D), jnp.float32),
    )(o0, o1, d0, d1)


def kernel(edge_index, user_embed, laplacian_lambda_max, cheb_W, cheb_b,
           gat_Wsrc, gat_bsrc, gat_Wdst, gat_bdst, gat_attn):
    src = edge_index[0]
    dst = edge_index[1]
    rn = (2.0 / laplacian_lambda_max).astype(jnp.float32)
    w0 = cheb_W[:D]
    w1 = cheb_W[D:2 * D]
    w2 = cheb_W[2 * D:]
    bias = cheb_b.reshape(1, D)
    bs = gat_bsrc.reshape(1, D)
    bd = gat_bdst.reshape(1, D)
    attn = gat_attn.reshape(D)

    zd = jnp.zeros((N, D), jnp.float32)
    z16 = jnp.zeros((N, L), jnp.float32)

    src3 = src.reshape(NW, NCHUNK, CH)
    dst3 = dst.reshape(NW, NCHUNK, CH)

    degp = _sc_deg(dst, z16)
    dinv, h = _tc_deg_h(degp, user_embed)

    def cheb(x0, h0):
        s = _sc_segsum(src3, dst3, h0, zd)
        x1, h1 = _tc_step1(s[0], s[1], x0, dinv, rn)
        s2 = _sc_segsum(src3, dst3, h1, zd)
        return _tc_step2(s2[0], s2[1], x0, x1, dinv, rn, w0, w1, w2, bias)

    x1, h1 = cheb(user_embed, h)
    x2, _ = cheb(x1, h1)

    fs, fd = _tc_fsfd(x2, gat_Wsrc, bs, gat_Wdst, bd)
    acc, den = _sc_gat(src, dst, fs, fd, attn, zd, z16)
    return _tc_final(acc[0], acc[1], den[0], den[1])


# speedup vs baseline: 9.0827x; 1.2446x over previous
import functools

import jax
import jax.numpy as jnp
from jax import lax
from jax.experimental import pallas as pl
from jax.experimental.pallas import tpu as pltpu
from jax.experimental.pallas import tpu_sc as plsc

N = 10000
E = 320000
D = 128

NC = 2
NS = 16
NW = NC * NS
L = 16

EPT = E // NW
CH = 80
NCHUNK = EPT // CH
RPT = N // NS

_mesh = plsc.VectorSubcoreMesh(core_axis_name="c", subcore_axis_name="s",
                               num_cores=NC, num_subcores=NS)
_sc_params = pltpu.CompilerParams(use_tc_tiling_on_sc=False,
                                  needs_layout_passes=False)


def _tile_base(unused=None):
    core = lax.axis_index("c")
    sub = lax.axis_index("s")
    return core, sub, (core * NS + sub) * EPT


def _sc_deg_body(dst_hbm, z16_hbm, out_hbm, acc_sh, idx_d, ones_v, sem):
    core, sub, base0 = _tile_base()

    pltpu.sync_copy(z16_hbm.at[pl.ds(sub * RPT, RPT)],
                    acc_sh.at[pl.ds(sub * RPT, RPT)])

    @pl.loop(0, CH)
    def _ones(i):
        ones_v[i, :] = jnp.ones((L,), jnp.float32)

    plsc.subcore_barrier()

    @pl.loop(0, NCHUNK)
    def _chunk(ci):
        base = pl.multiple_of(base0 + ci * CH, 8)
        pltpu.async_copy(dst_hbm.at[pl.ds(base, CH)], idx_d, sem).wait()
        pltpu.sync_copy(ones_v, acc_sh.at[idx_d], add=True)

    plsc.subcore_barrier()
    pltpu.sync_copy(acc_sh.at[pl.ds(sub * RPT, RPT)],
                    out_hbm.at[core, pl.ds(sub * RPT, RPT)])


@jax.jit
def _sc_deg(dst, z16):
    k = pl.kernel(
        _sc_deg_body,
        out_type=jax.ShapeDtypeStruct((NC, N, L), jnp.float32),
        mesh=_mesh,
        compiler_params=_sc_params,
        scratch_types=[
            pltpu.VMEM_SHARED((N, L), jnp.float32),
            pltpu.VMEM((CH,), jnp.int32),
            pltpu.VMEM((CH, L), jnp.float32),
            pltpu.SemaphoreType.DMA,
        ],
    )
    return k(dst, z16)


def _sc_segsum_body(src3_hbm, dst3_hbm, table_hbm, zd_hbm, out_hbm,
                    acc_sh, idxs, idxd, rows0, rows1, semi, sem0, sem1):
    core, sub, base0 = _tile_base()
    wid = core * NS + sub

    cs = pltpu.async_copy(src3_hbm.at[wid], idxs, semi)
    cd = pltpu.async_copy(dst3_hbm.at[wid], idxd, semi)
    pltpu.sync_copy(zd_hbm.at[pl.ds(sub * RPT, RPT)],
                    acc_sh.at[pl.ds(sub * RPT, RPT)])
    cs.wait()
    cd.wait()
    plsc.subcore_barrier()

    pltpu.async_copy(table_hbm.at[idxs.at[0]], rows0, sem0)

    @pl.loop(0, NCHUNK - 1, step=2)
    def _chunk(ci0):
        pltpu.make_async_copy(table_hbm.at[idxs.at[ci0]], rows0, sem0).wait()
        pltpu.async_copy(table_hbm.at[idxs.at[ci0 + 1]], rows1, sem1)
        pltpu.sync_copy(rows0, acc_sh.at[idxd.at[ci0]], add=True)
        pltpu.make_async_copy(table_hbm.at[idxs.at[ci0 + 1]], rows1, sem1).wait()
        pltpu.async_copy(table_hbm.at[idxs.at[ci0 + 2]], rows0, sem0)
        pltpu.sync_copy(rows1, acc_sh.at[idxd.at[ci0 + 1]], add=True)

    last = NCHUNK - 1
    pltpu.make_async_copy(table_hbm.at[idxs.at[last]], rows0, sem0).wait()
    pltpu.sync_copy(rows0, acc_sh.at[idxd.at[last]], add=True)

    plsc.subcore_barrier()
    for k in range(RPT // 125):
        r0 = sub * RPT + k * 125
        pltpu.sync_copy(acc_sh.at[pl.ds(r0, 125)], out_hbm.at[core, pl.ds(r0, 125)])


@jax.jit
def _sc_segsum(src3, dst3, table, zd):
    k = pl.kernel(
        _sc_segsum_body,
        out_type=jax.ShapeDtypeStruct((NC, N, D), jnp.float32),
        mesh=_mesh,
        compiler_params=_sc_params,
        scratch_types=[
            pltpu.VMEM_SHARED((N, D), jnp.float32),
            pltpu.VMEM((NCHUNK, CH), jnp.int32),
            pltpu.VMEM((NCHUNK, CH), jnp.int32),
            pltpu.VMEM((CH, D), jnp.float32),
            pltpu.VMEM((CH, D), jnp.float32),
            pltpu.SemaphoreType.DMA,
            pltpu.SemaphoreType.DMA,
            pltpu.SemaphoreType.DMA,
        ],
    )
    return k(src3, dst3, table, zd)


def _sc_gat_body(src3_hbm, dst3_hbm, fs_hbm, fd_hbm, attn_hbm, zd_hbm, z16_hbm,
                 out_hbm, den_hbm,
                 acc_sh, den_sh, is0, id0, is1, id1, a0, a1, b, drows,
                 attn_v, pmat, ebuf, semi0, semi1, sema0, sema1, semb):
    core, sub, base0 = _tile_base()
    wid = core * NS + sub

    ci0s = pltpu.async_copy(src3_hbm.at[wid, 0], is0, semi0)
    ci0d = pltpu.async_copy(dst3_hbm.at[wid, 0], id0, semi0)
    pltpu.sync_copy(zd_hbm.at[pl.ds(sub * RPT, RPT)],
                    acc_sh.at[pl.ds(sub * RPT, RPT)])
    pltpu.sync_copy(z16_hbm.at[pl.ds(sub * RPT, RPT)],
                    den_sh.at[pl.ds(sub * RPT, RPT)])
    pltpu.sync_copy(attn_hbm, attn_v)
    ci0s.wait()
    ci0d.wait()
    pltpu.async_copy(fs_hbm.at[is0], a0, sema0)
    pltpu.async_copy(fd_hbm.at[id0], b, semb)
    pltpu.async_copy(src3_hbm.at[wid, 1], is1, semi1)
    pltpu.async_copy(dst3_hbm.at[wid, 1], id1, semi1)
    plsc.subcore_barrier()

    lanes = lax.iota(jnp.int32, L)
    attn_c = [attn_v[pl.ds(j * L, L)] for j in range(D // L)]

    def lane_bcast(vec, i):
        idx = jnp.full((L, 1), i, jnp.int32)
        dnums = lax.GatherDimensionNumbers(
            offset_dims=(), collapsed_slice_dims=(0,), start_index_map=(0,))
        return lax.gather(vec, idx, dnums, (1,),
                          mode=lax.GatherScatterMode.PROMISE_IN_BOUNDS)

    def logits_phase(arows):
        @pl.loop(0, CH, step=L)
        def _grp(g):
            for i in range(L):
                acc = jnp.zeros((L,), jnp.float32)
                for j in range(D // L):
                    sv = arows[g + i, pl.ds(j * L, L)] + b[g + i, pl.ds(j * L, L)]
                    sv = jnp.where(sv >= 0.0, sv, sv * jnp.float32(0.2))
                    acc = acc + sv * attn_c[j]
                pmat[i, :] = acc
            e16 = jnp.zeros((L,), jnp.float32)
            for kk in range(L):
                e16 = e16 + plsc.load_gather(
                    pmat, [lanes, jnp.full((L,), kk, jnp.int32)])
            ebuf[pl.ds(g, L)] = jnp.exp(e16)

    def scale_phase(arows):
        @pl.loop(0, CH, step=L)
        def _grp(g):
            ee16 = ebuf[pl.ds(g, L)]
            for i in range(L):
                bc = lane_bcast(ee16, i)
                for j in range(D // L):
                    arows[g + i, pl.ds(j * L, L)] = (
                        arows[g + i, pl.ds(j * L, L)] * bc)
                drows[g + i, :] = bc

    def run_chunk(ci, arows, sema, ism, idm, iso, ido, semio, semim, last):
        pltpu.make_async_copy(fs_hbm.at[ism], arows, sema).wait()
        pltpu.make_async_copy(fd_hbm.at[idm], b, semb).wait()
        logits_phase(arows)
        if not last:
            pltpu.make_async_copy(src3_hbm.at[wid, ci + 1], iso, semio).wait()
            pltpu.make_async_copy(dst3_hbm.at[wid, ci + 1], ido, semio).wait()
            other_a = a1 if arows is a0 else a0
            other_sema = sema1 if arows is a0 else sema0
            pltpu.async_copy(fs_hbm.at[iso], other_a, other_sema)
            pltpu.async_copy(fd_hbm.at[ido], b, semb)
        scale_phase(arows)
        pltpu.sync_copy(arows, acc_sh.at[idm], add=True)
        pltpu.sync_copy(drows, den_sh.at[idm], add=True)

    @pl.loop(0, NCHUNK - 1, step=2)
    def _pair(ci0):
        run_chunk(ci0, a0, sema0, is0, id0, is1, id1, semi1, semi0, False)

        @pl.when(ci0 + 2 < NCHUNK)
        def _():
            pltpu.async_copy(src3_hbm.at[wid, ci0 + 2], is0, semi0)
            pltpu.async_copy(dst3_hbm.at[wid, ci0 + 2], id0, semi0)

        run_chunk(ci0 + 1, a1, sema1, is1, id1, is0, id0, semi0, semi1, False)

        @pl.when(ci0 + 3 < NCHUNK)
        def _():
            pltpu.async_copy(src3_hbm.at[wid, ci0 + 3], is1, semi1)
            pltpu.async_copy(dst3_hbm.at[wid, ci0 + 3], id1, semi1)

    run_chunk(NCHUNK - 1, a0, sema0, is0, id0, is1, id1, semi1, semi0, True)

    plsc.subcore_barrier()
    for k in range(RPT // 125):
        r0 = sub * RPT + k * 125
        pltpu.sync_copy(acc_sh.at[pl.ds(r0, 125)], out_hbm.at[core, pl.ds(r0, 125)])
    pltpu.sync_copy(den_sh.at[pl.ds(sub * RPT, RPT)],
                    den_hbm.at[core, pl.ds(sub * RPT, RPT)])


@jax.jit
def _sc_gat(src3, dst3, fs, fd, attn, zd, z16):
    k = pl.kernel(
        _sc_gat_body,
        out_type=(jax.ShapeDtypeStruct((NC, N, D), jnp.float32),
                  jax.ShapeDtypeStruct((NC, N, L), jnp.float32)),
        mesh=_mesh,
        compiler_params=_sc_params,
        scratch_types=[
            pltpu.VMEM_SHARED((N, D), jnp.float32),
            pltpu.VMEM_SHARED((N, L), jnp.float32),
            pltpu.VMEM((CH,), jnp.int32),
            pltpu.VMEM((CH,), jnp.int32),
            pltpu.VMEM((CH,), jnp.int32),
            pltpu.VMEM((CH,), jnp.int32),
            pltpu.VMEM((CH, D), jnp.float32),
            pltpu.VMEM((CH, D), jnp.float32),
            pltpu.VMEM((CH, D), jnp.float32),
            pltpu.VMEM((CH, L), jnp.float32),
            pltpu.VMEM((D,), jnp.float32),
            pltpu.VMEM((L, L), jnp.float32),
            pltpu.VMEM((CH,), jnp.float32),
            pltpu.SemaphoreType.DMA,
            pltpu.SemaphoreType.DMA,
            pltpu.SemaphoreType.DMA,
            pltpu.SemaphoreType.DMA,
            pltpu.SemaphoreType.DMA,
        ],
    )
    return k(src3, dst3, fs, fd, attn, zd, z16)


BLK = 1000
GRID = N // BLK


def _row_spec(width=D):
    return pl.BlockSpec((BLK, width), lambda i: (i, 0))


def _full_spec(shape):
    ndim = len(shape)
    return pl.BlockSpec(shape, lambda i: (0,) * ndim)


def _tc_deg_h_body(d0_ref, d1_ref, x_ref, dinv_ref, h_ref):
    deg = d0_ref[:, 0:1] + d1_ref[:, 0:1]
    dinv = lax.rsqrt(jnp.maximum(deg, 1.0))
    dinv_ref[...] = jnp.broadcast_to(dinv, (BLK, L))
    h_ref[...] = x_ref[...] * dinv


@jax.jit
def _tc_deg_h(degp, x0):
    return pl.pallas_call(
        _tc_deg_h_body,
        grid=(GRID,),
        in_specs=[_row_spec(L), _row_spec(L), _row_spec()],
        out_specs=[_row_spec(L), _row_spec()],
        out_shape=[jax.ShapeDtypeStruct((N, L), jnp.float32),
                   jax.ShapeDtypeStruct((N, D), jnp.float32)],
    )(degp[0], degp[1], x0)


def _tc_step1_body(s0_ref, s1_ref, x0_ref, dinv_ref, rn_ref, x1_ref, h1_ref):
    rn = rn_ref[0]
    dv = dinv_ref[:, 0:1]
    agg = (s0_ref[...] + s1_ref[...]) * dv
    x1 = agg * (-rn) + x0_ref[...] * (rn - 1.0)
    x1_ref[...] = x1
    h1_ref[...] = x1 * dv


@jax.jit
def _tc_step1(s0, s1, x0, dinv, rn):
    return pl.pallas_call(
        _tc_step1_body,
        grid=(GRID,),
        in_specs=[_row_spec(), _row_spec(), _row_spec(), _row_spec(L),
                  pl.BlockSpec(memory_space=pltpu.SMEM)],
        out_specs=[_row_spec(), _row_spec()],
        out_shape=[jax.ShapeDtypeStruct((N, D), jnp.float32),
                   jax.ShapeDtypeStruct((N, D), jnp.float32)],
    )(s0, s1, x0, dinv, rn)


def _tc_step2_body(s0_ref, s1_ref, x0_ref, x1_ref, dinv_ref, rn_ref,
                   w0_ref, w1_ref, w2_ref, b_ref, x_ref, h_ref):
    rn = rn_ref[0]
    dv = dinv_ref[:, 0:1]
    agg = (s0_ref[...] + s1_ref[...]) * dv
    x2 = agg * (-2.0 * rn) + x1_ref[...] * (2.0 * (rn - 1.0)) - x0_ref[...]
    acc = jnp.dot(x0_ref[...], w0_ref[...],
                  preferred_element_type=jnp.float32,
                  precision=lax.Precision.HIGHEST)
    acc += jnp.dot(x1_ref[...], w1_ref[...],
                   preferred_element_type=jnp.float32,
                  precision=lax.Precision.HIGHEST)
    acc += jnp.dot(x2, w2_ref[...], preferred_element_type=jnp.float32,
                  precision=lax.Precision.HIGHEST)
    xn = jnp.maximum(acc + b_ref[...], 0.0)
    x_ref[...] = xn
    h_ref[...] = xn * dv


@jax.jit
def _tc_step2(s0, s1, x0, x1, dinv, rn, w0, w1, w2, b):
    return pl.pallas_call(
        _tc_step2_body,
        grid=(GRID,),
        in_specs=[_row_spec(), _row_spec(), _row_spec(), _row_spec(),
                  _row_spec(L),
                  pl.BlockSpec(memory_space=pltpu.SMEM),
                  _full_spec((D, D)), _full_spec((D, D)), _full_spec((D, D)),
                  _full_spec((1, D))],
        out_specs=[_row_spec(), _row_spec()],
        out_shape=[jax.ShapeDtypeStruct((N, D), jnp.float32),
                   jax.ShapeDtypeStruct((N, D), jnp.float32)],
    )(s0, s1, x0, x1, dinv, rn, w0, w1, w2, b)


def _tc_fsfd_body(x_ref, ws_ref, bs_ref, wd_ref, bd_ref, fs_ref, fd_ref):
    x = x_ref[...]
    fs_ref[...] = jnp.dot(x, ws_ref[...],
                          preferred_element_type=jnp.float32,
                  precision=lax.Precision.HIGHEST) + bs_ref[...]
    fd_ref[...] = jnp.dot(x, wd_ref[...],
                          preferred_element_type=jnp.float32,
                  precision=lax.Precision.HIGHEST) + bd_ref[...]


@jax.jit
def _tc_fsfd(x, ws, bs, wd, bd):
    return pl.pallas_call(
        _tc_fsfd_body,
        grid=(GRID,),
        in_specs=[_row_spec(), _full_spec((D, D)), _full_spec((1, D)),
                  _full_spec((D, D)), _full_spec((1, D))],
        out_specs=[_row_spec(), _row_spec()],
        out_shape=[jax.ShapeDtypeStruct((N, D), jnp.float32),
                   jax.ShapeDtypeStruct((N, D), jnp.float32)],
    )(x, ws, bs, wd, bd)


def _tc_final_body(o0_ref, o1_ref, d0_ref, d1_ref, out_ref):
    den = d0_ref[:, 0:1] + d1_ref[:, 0:1]
    num = o0_ref[...] + o1_ref[...]
    out_ref[...] = jnp.where(den > 0.0, num / jnp.maximum(den, 1e-38), 0.0)


@jax.jit
def _tc_final(o0, o1, d0, d1):
    return pl.pallas_call(
        _tc_final_body,
        grid=(GRID,),
        in_specs=[_row_spec(), _row_spec(), _row_spec(L), _row_spec(L)],
        out_specs=_row_spec(),
        out_shape=jax.ShapeDtypeStruct((N, D), jnp.float32),
    )(o0, o1, d0, d1)


def kernel(edge_index, user_embed, laplacian_lambda_max, cheb_W, cheb_b,
           gat_Wsrc, gat_bsrc, gat_Wdst, gat_bdst, gat_attn):
    src = edge_index[0]
    dst = edge_index[1]
    rn = (2.0 / laplacian_lambda_max).astype(jnp.float32)
    w0 = cheb_W[:D]
    w1 = cheb_W[D:2 * D]
    w2 = cheb_W[2 * D:]
    bias = cheb_b.reshape(1, D)
    bs = gat_bsrc.reshape(1, D)
    bd = gat_bdst.reshape(1, D)
    attn = gat_attn.reshape(D)

    zd = jnp.zeros((N, D), jnp.float32)
    z16 = jnp.zeros((N, L), jnp.float32)

    src3 = src.reshape(NW, NCHUNK, CH)
    dst3 = dst.reshape(NW, NCHUNK, CH)

    degp = _sc_deg(dst, z16)
    dinv, h = _tc_deg_h(degp, user_embed)

    def cheb(x0, h0):
        s = _sc_segsum(src3, dst3, h0, zd)
        x1, h1 = _tc_step1(s[0], s[1], x0, dinv, rn)
        s2 = _sc_segsum(src3, dst3, h1, zd)
        return _tc_step2(s2[0], s2[1], x0, x1, dinv, rn, w0, w1, w2, bias)

    x1, h1 = cheb(user_embed, h)
    x2, _ = cheb(x1, h1)

    fs, fd = _tc_fsfd(x2, gat_Wsrc, bs, gat_Wdst, bd)
    acc, den = _sc_gat(src3, dst3, fs, fd, attn, zd, z16)
    return _tc_final(acc[0], acc[1], den[0], den[1])


# speedup vs baseline: 9.4725x; 1.0429x over previous
import functools

import jax
import jax.numpy as jnp
from jax import lax
from jax.experimental import pallas as pl
from jax.experimental.pallas import tpu as pltpu
from jax.experimental.pallas import tpu_sc as plsc

N = 10000
E = 320000
D = 128

NC = 2
NS = 16
NW = NC * NS
L = 16

EPT = E // NW
CH = 80
NCHUNK = EPT // CH
RPT = N // NS

_mesh = plsc.VectorSubcoreMesh(core_axis_name="c", subcore_axis_name="s",
                               num_cores=NC, num_subcores=NS)
_sc_params = pltpu.CompilerParams(use_tc_tiling_on_sc=False,
                                  needs_layout_passes=False)


def _tile_base(unused=None):
    core = lax.axis_index("c")
    sub = lax.axis_index("s")
    return core, sub, (core * NS + sub) * EPT


def _sc_deg_body(dst3_hbm, z16_hbm, out_hbm, acc_sh, idxd, ones_v, semi):
    core, sub, base0 = _tile_base()
    wid = core * NS + sub

    cd = pltpu.async_copy(dst3_hbm.at[wid], idxd, semi)
    pltpu.sync_copy(z16_hbm.at[pl.ds(sub * RPT, RPT)],
                    acc_sh.at[pl.ds(sub * RPT, RPT)])

    @pl.loop(0, CH)
    def _ones(i):
        ones_v[i, :] = jnp.ones((L,), jnp.float32)

    cd.wait()
    plsc.subcore_barrier()

    @pl.loop(0, NCHUNK)
    def _chunk(ci):
        pltpu.sync_copy(ones_v, acc_sh.at[idxd.at[ci]], add=True)

    plsc.subcore_barrier()
    pltpu.sync_copy(acc_sh.at[pl.ds(sub * RPT, RPT)],
                    out_hbm.at[core, pl.ds(sub * RPT, RPT)])


@jax.jit
def _sc_deg(dst3, z16):
    k = pl.kernel(
        _sc_deg_body,
        out_type=jax.ShapeDtypeStruct((NC, N, L), jnp.float32),
        mesh=_mesh,
        compiler_params=_sc_params,
        scratch_types=[
            pltpu.VMEM_SHARED((N, L), jnp.float32),
            pltpu.VMEM((NCHUNK, CH), jnp.int32),
            pltpu.VMEM((CH, L), jnp.float32),
            pltpu.SemaphoreType.DMA,
        ],
    )
    return k(dst3, z16)


def _sc_segsum_body(src3_hbm, dst3_hbm, table_hbm, zd_hbm, out_hbm,
                    acc_sh, idxs, idxd, rows0, rows1, semi, sem0, sem1):
    core, sub, base0 = _tile_base()
    wid = core * NS + sub

    cs = pltpu.async_copy(src3_hbm.at[wid], idxs, semi)
    cd = pltpu.async_copy(dst3_hbm.at[wid], idxd, semi)
    pltpu.sync_copy(zd_hbm.at[pl.ds(sub * RPT, RPT)],
                    acc_sh.at[pl.ds(sub * RPT, RPT)])
    cs.wait()
    cd.wait()
    plsc.subcore_barrier()

    pltpu.async_copy(table_hbm.at[idxs.at[0]], rows0, sem0)

    @pl.loop(0, NCHUNK - 1, step=2)
    def _chunk(ci0):
        pltpu.make_async_copy(table_hbm.at[idxs.at[ci0]], rows0, sem0).wait()
        pltpu.async_copy(table_hbm.at[idxs.at[ci0 + 1]], rows1, sem1)
        pltpu.sync_copy(rows0, acc_sh.at[idxd.at[ci0]], add=True)
        pltpu.make_async_copy(table_hbm.at[idxs.at[ci0 + 1]], rows1, sem1).wait()
        pltpu.async_copy(table_hbm.at[idxs.at[ci0 + 2]], rows0, sem0)
        pltpu.sync_copy(rows1, acc_sh.at[idxd.at[ci0 + 1]], add=True)

    last = NCHUNK - 1
    pltpu.make_async_copy(table_hbm.at[idxs.at[last]], rows0, sem0).wait()
    pltpu.sync_copy(rows0, acc_sh.at[idxd.at[last]], add=True)

    plsc.subcore_barrier()
    for k in range(RPT // 125):
        r0 = sub * RPT + k * 125
        pltpu.sync_copy(acc_sh.at[pl.ds(r0, 125)], out_hbm.at[core, pl.ds(r0, 125)])


@jax.jit
def _sc_segsum(src3, dst3, table, zd):
    k = pl.kernel(
        _sc_segsum_body,
        out_type=jax.ShapeDtypeStruct((NC, N, D), jnp.float32),
        mesh=_mesh,
        compiler_params=_sc_params,
        scratch_types=[
            pltpu.VMEM_SHARED((N, D), jnp.float32),
            pltpu.VMEM((NCHUNK, CH), jnp.int32),
            pltpu.VMEM((NCHUNK, CH), jnp.int32),
            pltpu.VMEM((CH, D), jnp.float32),
            pltpu.VMEM((CH, D), jnp.float32),
            pltpu.SemaphoreType.DMA,
            pltpu.SemaphoreType.DMA,
            pltpu.SemaphoreType.DMA,
        ],
    )
    return k(src3, dst3, table, zd)


def _sc_gat_body(src3_hbm, dst3_hbm, fs_hbm, fd_hbm, attn_hbm, zd_hbm, z16_hbm,
                 out_hbm, den_hbm,
                 acc_sh, den_sh, is0, id0, is1, id1, a0, a1, b, drows,
                 attn_v, pmat, ebuf, semi0, semi1, sema0, sema1, semb):
    core, sub, base0 = _tile_base()
    wid = core * NS + sub

    ci0s = pltpu.async_copy(src3_hbm.at[wid, 0], is0, semi0)
    ci0d = pltpu.async_copy(dst3_hbm.at[wid, 0], id0, semi0)
    pltpu.sync_copy(zd_hbm.at[pl.ds(sub * RPT, RPT)],
                    acc_sh.at[pl.ds(sub * RPT, RPT)])
    pltpu.sync_copy(z16_hbm.at[pl.ds(sub * RPT, RPT)],
                    den_sh.at[pl.ds(sub * RPT, RPT)])
    pltpu.sync_copy(attn_hbm, attn_v)
    ci0s.wait()
    ci0d.wait()
    pltpu.async_copy(fs_hbm.at[is0], a0, sema0)
    pltpu.async_copy(fd_hbm.at[id0], b, semb)
    pltpu.async_copy(src3_hbm.at[wid, 1], is1, semi1)
    pltpu.async_copy(dst3_hbm.at[wid, 1], id1, semi1)
    plsc.subcore_barrier()

    lanes = lax.iota(jnp.int32, L)
    attn_c = [attn_v[pl.ds(j * L, L)] for j in range(D // L)]

    def lane_bcast(vec, i):
        idx = jnp.full((L, 1), i, jnp.int32)
        dnums = lax.GatherDimensionNumbers(
            offset_dims=(), collapsed_slice_dims=(0,), start_index_map=(0,))
        return lax.gather(vec, idx, dnums, (1,),
                          mode=lax.GatherScatterMode.PROMISE_IN_BOUNDS)

    def logits_phase(arows):
        @pl.loop(0, CH, step=L)
        def _grp(g):
            for i in range(L):
                acc = jnp.zeros((L,), jnp.float32)
                for j in range(D // L):
                    sv = arows[g + i, pl.ds(j * L, L)] + b[g + i, pl.ds(j * L, L)]
                    sv = jnp.where(sv >= 0.0, sv, sv * jnp.float32(0.2))
                    acc = acc + sv * attn_c[j]
                pmat[i, :] = acc
            e16 = jnp.zeros((L,), jnp.float32)
            for kk in range(L):
                e16 = e16 + plsc.load_gather(
                    pmat, [lanes, jnp.full((L,), kk, jnp.int32)])
            ebuf[pl.ds(g, L)] = jnp.exp(e16)

    def scale_phase(arows):
        @pl.loop(0, CH, step=L)
        def _grp(g):
            ee16 = ebuf[pl.ds(g, L)]
            for i in range(L):
                bc = lane_bcast(ee16, i)
                for j in range(D // L):
                    arows[g + i, pl.ds(j * L, L)] = (
                        arows[g + i, pl.ds(j * L, L)] * bc)
                drows[g + i, :] = bc

    def run_chunk(ci, arows, sema, ism, idm, iso, ido, semio, semim, last):
        pltpu.make_async_copy(fs_hbm.at[ism], arows, sema).wait()
        pltpu.make_async_copy(fd_hbm.at[idm], b, semb).wait()
        logits_phase(arows)
        if not last:
            pltpu.make_async_copy(src3_hbm.at[wid, ci + 1], iso, semio).wait()
            pltpu.make_async_copy(dst3_hbm.at[wid, ci + 1], ido, semio).wait()
            other_a = a1 if arows is a0 else a0
            other_sema = sema1 if arows is a0 else sema0
            pltpu.async_copy(fs_hbm.at[iso], other_a, other_sema)
            pltpu.async_copy(fd_hbm.at[ido], b, semb)
        scale_phase(arows)
        pltpu.sync_copy(arows, acc_sh.at[idm], add=True)
        pltpu.sync_copy(drows, den_sh.at[idm], add=True)

    @pl.loop(0, NCHUNK - 1, step=2)
    def _pair(ci0):
        run_chunk(ci0, a0, sema0, is0, id0, is1, id1, semi1, semi0, False)

        @pl.when(ci0 + 2 < NCHUNK)
        def _():
            pltpu.async_copy(src3_hbm.at[wid, ci0 + 2], is0, semi0)
            pltpu.async_copy(dst3_hbm.at[wid, ci0 + 2], id0, semi0)

        run_chunk(ci0 + 1, a1, sema1, is1, id1, is0, id0, semi0, semi1, False)

        @pl.when(ci0 + 3 < NCHUNK)
        def _():
            pltpu.async_copy(src3_hbm.at[wid, ci0 + 3], is1, semi1)
            pltpu.async_copy(dst3_hbm.at[wid, ci0 + 3], id1, semi1)

    run_chunk(NCHUNK - 1, a0, sema0, is0, id0, is1, id1, semi1, semi0, True)

    plsc.subcore_barrier()
    for k in range(RPT // 125):
        r0 = sub * RPT + k * 125
        pltpu.sync_copy(acc_sh.at[pl.ds(r0, 125)], out_hbm.at[core, pl.ds(r0, 125)])
    pltpu.sync_copy(den_sh.at[pl.ds(sub * RPT, RPT)],
                    den_hbm.at[core, pl.ds(sub * RPT, RPT)])


@jax.jit
def _sc_gat(src3, dst3, fs, fd, attn, zd, z16):
    k = pl.kernel(
        _sc_gat_body,
        out_type=(jax.ShapeDtypeStruct((NC, N, D), jnp.float32),
                  jax.ShapeDtypeStruct((NC, N, L), jnp.float32)),
        mesh=_mesh,
        compiler_params=_sc_params,
        scratch_types=[
            pltpu.VMEM_SHARED((N, D), jnp.float32),
            pltpu.VMEM_SHARED((N, L), jnp.float32),
            pltpu.VMEM((CH,), jnp.int32),
            pltpu.VMEM((CH,), jnp.int32),
            pltpu.VMEM((CH,), jnp.int32),
            pltpu.VMEM((CH,), jnp.int32),
            pltpu.VMEM((CH, D), jnp.float32),
            pltpu.VMEM((CH, D), jnp.float32),
            pltpu.VMEM((CH, D), jnp.float32),
            pltpu.VMEM((CH, L), jnp.float32),
            pltpu.VMEM((D,), jnp.float32),
            pltpu.VMEM((L, L), jnp.float32),
            pltpu.VMEM((CH,), jnp.float32),
            pltpu.SemaphoreType.DMA,
            pltpu.SemaphoreType.DMA,
            pltpu.SemaphoreType.DMA,
            pltpu.SemaphoreType.DMA,
            pltpu.SemaphoreType.DMA,
        ],
    )
    return k(src3, dst3, fs, fd, attn, zd, z16)


BLK = 1000
GRID = N // BLK


def _row_spec(width=D):
    return pl.BlockSpec((BLK, width), lambda i: (i, 0))


def _full_spec(shape):
    ndim = len(shape)
    return pl.BlockSpec(shape, lambda i: (0,) * ndim)


def _tc_deg_h_body(d0_ref, d1_ref, x_ref, dinv_ref, h_ref):
    deg = d0_ref[:, 0:1] + d1_ref[:, 0:1]
    dinv = lax.rsqrt(jnp.maximum(deg, 1.0))
    dinv_ref[...] = jnp.broadcast_to(dinv, (BLK, L))
    h_ref[...] = x_ref[...] * dinv


@jax.jit
def _tc_deg_h(degp, x0):
    return pl.pallas_call(
        _tc_deg_h_body,
        grid=(GRID,),
        in_specs=[_row_spec(L), _row_spec(L), _row_spec()],
        out_specs=[_row_spec(L), _row_spec()],
        out_shape=[jax.ShapeDtypeStruct((N, L), jnp.float32),
                   jax.ShapeDtypeStruct((N, D), jnp.float32)],
    )(degp[0], degp[1], x0)


def _tc_step1_body(s0_ref, s1_ref, x0_ref, dinv_ref, rn_ref, x1_ref, h1_ref):
    rn = rn_ref[0]
    dv = dinv_ref[:, 0:1]
    agg = (s0_ref[...] + s1_ref[...]) * dv
    x1 = agg * (-rn) + x0_ref[...] * (rn - 1.0)
    x1_ref[...] = x1
    h1_ref[...] = x1 * dv


@jax.jit
def _tc_step1(s0, s1, x0, dinv, rn):
    return pl.pallas_call(
        _tc_step1_body,
        grid=(GRID,),
        in_specs=[_row_spec(), _row_spec(), _row_spec(), _row_spec(L),
                  pl.BlockSpec(memory_space=pltpu.SMEM)],
        out_specs=[_row_spec(), _row_spec()],
        out_shape=[jax.ShapeDtypeStruct((N, D), jnp.float32),
                   jax.ShapeDtypeStruct((N, D), jnp.float32)],
    )(s0, s1, x0, dinv, rn)


def _tc_step2_body(s0_ref, s1_ref, x0_ref, x1_ref, dinv_ref, rn_ref,
                   w0_ref, w1_ref, w2_ref, b_ref, x_ref, h_ref):
    rn = rn_ref[0]
    dv = dinv_ref[:, 0:1]
    agg = (s0_ref[...] + s1_ref[...]) * dv
    x2 = agg * (-2.0 * rn) + x1_ref[...] * (2.0 * (rn - 1.0)) - x0_ref[...]
    acc = jnp.dot(x0_ref[...], w0_ref[...],
                  preferred_element_type=jnp.float32,
                  precision=lax.Precision.HIGHEST)
    acc += jnp.dot(x1_ref[...], w1_ref[...],
                   preferred_element_type=jnp.float32,
                  precision=lax.Precision.HIGHEST)
    acc += jnp.dot(x2, w2_ref[...], preferred_element_type=jnp.float32,
                  precision=lax.Precision.HIGHEST)
    xn = jnp.maximum(acc + b_ref[...], 0.0)
    x_ref[...] = xn
    h_ref[...] = xn * dv


@jax.jit
def _tc_step2(s0, s1, x0, x1, dinv, rn, w0, w1, w2, b):
    return pl.pallas_call(
        _tc_step2_body,
        grid=(GRID,),
        in_specs=[_row_spec(), _row_spec(), _row_spec(), _row_spec(),
                  _row_spec(L),
                  pl.BlockSpec(memory_space=pltpu.SMEM),
                  _full_spec((D, D)), _full_spec((D, D)), _full_spec((D, D)),
                  _full_spec((1, D))],
        out_specs=[_row_spec(), _row_spec()],
        out_shape=[jax.ShapeDtypeStruct((N, D), jnp.float32),
                   jax.ShapeDtypeStruct((N, D), jnp.float32)],
    )(s0, s1, x0, x1, dinv, rn, w0, w1, w2, b)


def _tc_step2gat_body(s0_ref, s1_ref, x0_ref, x1_ref, dinv_ref, rn_ref,
                      w0_ref, w1_ref, w2_ref, b_ref,
                      ws_ref, bs_ref, wd_ref, bd_ref, fs_ref, fd_ref):
    rn = rn_ref[0]
    dv = dinv_ref[:, 0:1]
    agg = (s0_ref[...] + s1_ref[...]) * dv
    x2 = agg * (-2.0 * rn) + x1_ref[...] * (2.0 * (rn - 1.0)) - x0_ref[...]
    acc = jnp.dot(x0_ref[...], w0_ref[...],
                  preferred_element_type=jnp.float32,
                  precision=lax.Precision.HIGHEST)
    acc += jnp.dot(x1_ref[...], w1_ref[...],
                   preferred_element_type=jnp.float32,
                   precision=lax.Precision.HIGHEST)
    acc += jnp.dot(x2, w2_ref[...], preferred_element_type=jnp.float32,
                   precision=lax.Precision.HIGHEST)
    xn = jnp.maximum(acc + b_ref[...], 0.0)
    fs_ref[...] = jnp.dot(xn, ws_ref[...],
                          preferred_element_type=jnp.float32,
                          precision=lax.Precision.HIGHEST) + bs_ref[...]
    fd_ref[...] = jnp.dot(xn, wd_ref[...],
                          preferred_element_type=jnp.float32,
                          precision=lax.Precision.HIGHEST) + bd_ref[...]


@jax.jit
def _tc_step2gat(s0, s1, x0, x1, dinv, rn, w0, w1, w2, b, ws, bs, wd, bd):
    return pl.pallas_call(
        _tc_step2gat_body,
        grid=(GRID,),
        in_specs=[_row_spec(), _row_spec(), _row_spec(), _row_spec(),
                  _row_spec(L),
                  pl.BlockSpec(memory_space=pltpu.SMEM),
                  _full_spec((D, D)), _full_spec((D, D)), _full_spec((D, D)),
                  _full_spec((1, D)),
                  _full_spec((D, D)), _full_spec((1, D)),
                  _full_spec((D, D)), _full_spec((1, D))],
        out_specs=[_row_spec(), _row_spec()],
        out_shape=[jax.ShapeDtypeStruct((N, D), jnp.float32),
                   jax.ShapeDtypeStruct((N, D), jnp.float32)],
    )(s0, s1, x0, x1, dinv, rn, w0, w1, w2, b, ws, bs, wd, bd)


def _tc_fsfd_body(x_ref, ws_ref, bs_ref, wd_ref, bd_ref, fs_ref, fd_ref):
    x = x_ref[...]
    fs_ref[...] = jnp.dot(x, ws_ref[...],
                          preferred_element_type=jnp.float32,
                  precision=lax.Precision.HIGHEST) + bs_ref[...]
    fd_ref[...] = jnp.dot(x, wd_ref[...],
                          preferred_element_type=jnp.float32,
                  precision=lax.Precision.HIGHEST) + bd_ref[...]


@jax.jit
def _tc_fsfd(x, ws, bs, wd, bd):
    return pl.pallas_call(
        _tc_fsfd_body,
        grid=(GRID,),
        in_specs=[_row_spec(), _full_spec((D, D)), _full_spec((1, D)),
                  _full_spec((D, D)), _full_spec((1, D))],
        out_specs=[_row_spec(), _row_spec()],
        out_shape=[jax.ShapeDtypeStruct((N, D), jnp.float32),
                   jax.ShapeDtypeStruct((N, D), jnp.float32)],
    )(x, ws, bs, wd, bd)


def _tc_final_body(o0_ref, o1_ref, d0_ref, d1_ref, out_ref):
    den = d0_ref[:, 0:1] + d1_ref[:, 0:1]
    num = o0_ref[...] + o1_ref[...]
    out_ref[...] = jnp.where(den > 0.0, num / jnp.maximum(den, 1e-38), 0.0)


@jax.jit
def _tc_final(o0, o1, d0, d1):
    return pl.pallas_call(
        _tc_final_body,
        grid=(GRID,),
        in_specs=[_row_spec(), _row_spec(), _row_spec(L), _row_spec(L)],
        out_specs=_row_spec(),
        out_shape=jax.ShapeDtypeStruct((N, D), jnp.float32),
    )(o0, o1, d0, d1)


def kernel(edge_index, user_embed, laplacian_lambda_max, cheb_W, cheb_b,
           gat_Wsrc, gat_bsrc, gat_Wdst, gat_bdst, gat_attn):
    src = edge_index[0]
    dst = edge_index[1]
    rn = (2.0 / laplacian_lambda_max).astype(jnp.float32)
    w0 = cheb_W[:D]
    w1 = cheb_W[D:2 * D]
    w2 = cheb_W[2 * D:]
    bias = cheb_b.reshape(1, D)
    bs = gat_bsrc.reshape(1, D)
    bd = gat_bdst.reshape(1, D)
    attn = gat_attn.reshape(D)

    zd = jnp.zeros((N, D), jnp.float32)
    z16 = jnp.zeros((N, L), jnp.float32)

    src3 = src.reshape(NW, NCHUNK, CH)
    dst3 = dst.reshape(NW, NCHUNK, CH)

    degp = _sc_deg(dst3, z16)
    dinv, h = _tc_deg_h(degp, user_embed)

    def half_cheb(x0, h0):
        s = _sc_segsum(src3, dst3, h0, zd)
        x1, h1 = _tc_step1(s[0], s[1], x0, dinv, rn)
        s2 = _sc_segsum(src3, dst3, h1, zd)
        return s2, x1

    s2a, x1a = half_cheb(user_embed, h)
    x1, h1 = _tc_step2(s2a[0], s2a[1], user_embed, x1a, dinv, rn,
                       w0, w1, w2, bias)
    s2b, x1b = half_cheb(x1, h1)
    fs, fd = _tc_step2gat(s2b[0], s2b[1], x1, x1b, dinv, rn,
                          w0, w1, w2, bias, gat_Wsrc, bs, gat_Wdst, bd)
    acc, den = _sc_gat(src3, dst3, fs, fd, attn, zd, z16)
    return _tc_final(acc[0], acc[1], den[0], den[1])


# speedup vs baseline: 9.9318x; 1.0485x over previous
import functools

import jax
import jax.numpy as jnp
from jax import lax
from jax.experimental import pallas as pl
from jax.experimental.pallas import tpu as pltpu
from jax.experimental.pallas import tpu_sc as plsc

N = 10000
E = 320000
D = 128

NC = 2
NS = 16
NW = NC * NS
L = 16

EPT = E // NW
CH = 80
NCHUNK = EPT // CH
RPT = N // NS
SCH = 100
SNCH = EPT // SCH

_mesh = plsc.VectorSubcoreMesh(core_axis_name="c", subcore_axis_name="s",
                               num_cores=NC, num_subcores=NS)
_sc_params = pltpu.CompilerParams(use_tc_tiling_on_sc=False,
                                  needs_layout_passes=False)


def _tile_base(unused=None):
    core = lax.axis_index("c")
    sub = lax.axis_index("s")
    return core, sub, (core * NS + sub) * EPT


def _sc_deg_body(dst3_hbm, z16_hbm, out_hbm, acc_sh, idxd, ones_v, semi):
    core, sub, base0 = _tile_base()
    wid = core * NS + sub

    cd = pltpu.async_copy(dst3_hbm.at[wid], idxd, semi)
    pltpu.sync_copy(z16_hbm.at[pl.ds(sub * RPT, RPT)],
                    acc_sh.at[pl.ds(sub * RPT, RPT)])

    @pl.loop(0, CH)
    def _ones(i):
        ones_v[i, :] = jnp.ones((L,), jnp.float32)

    cd.wait()
    plsc.subcore_barrier()

    @pl.loop(0, NCHUNK)
    def _chunk(ci):
        pltpu.sync_copy(ones_v, acc_sh.at[idxd.at[ci]], add=True)

    plsc.subcore_barrier()
    pltpu.sync_copy(acc_sh.at[pl.ds(sub * RPT, RPT)],
                    out_hbm.at[core, pl.ds(sub * RPT, RPT)])


@jax.jit
def _sc_deg(dst3, z16):
    k = pl.kernel(
        _sc_deg_body,
        out_type=jax.ShapeDtypeStruct((NC, N, L), jnp.float32),
        mesh=_mesh,
        compiler_params=_sc_params,
        scratch_types=[
            pltpu.VMEM_SHARED((N, L), jnp.float32),
            pltpu.VMEM((NCHUNK, CH), jnp.int32),
            pltpu.VMEM((CH, L), jnp.float32),
            pltpu.SemaphoreType.DMA,
        ],
    )
    return k(dst3, z16)


def _sc_segsum_body(src3_hbm, dst3_hbm, table_hbm, zd_hbm, out_hbm,
                    acc_sh, idxs, idxd, rows0, rows1, semi, sem0, sem1):
    core, sub, base0 = _tile_base()
    wid = core * NS + sub

    cs = pltpu.async_copy(src3_hbm.at[wid], idxs, semi)
    cd = pltpu.async_copy(dst3_hbm.at[wid], idxd, semi)
    pltpu.sync_copy(zd_hbm.at[pl.ds(sub * RPT, RPT)],
                    acc_sh.at[pl.ds(sub * RPT, RPT)])
    cs.wait()
    cd.wait()
    plsc.subcore_barrier()

    pltpu.async_copy(table_hbm.at[idxs.at[0]], rows0, sem0)

    @pl.loop(0, SNCH, step=2)
    def _chunk(ci0):
        pltpu.make_async_copy(table_hbm.at[idxs.at[ci0]], rows0, sem0).wait()
        pltpu.async_copy(table_hbm.at[idxs.at[ci0 + 1]], rows1, sem1)
        pltpu.sync_copy(rows0, acc_sh.at[idxd.at[ci0]], add=True)
        pltpu.make_async_copy(table_hbm.at[idxs.at[ci0 + 1]], rows1, sem1).wait()

        @pl.when(ci0 + 2 < SNCH)
        def _():
            pltpu.async_copy(table_hbm.at[idxs.at[ci0 + 2]], rows0, sem0)

        pltpu.sync_copy(rows1, acc_sh.at[idxd.at[ci0 + 1]], add=True)

    plsc.subcore_barrier()
    for k in range(RPT // 125):
        r0 = sub * RPT + k * 125
        pltpu.sync_copy(acc_sh.at[pl.ds(r0, 125)], out_hbm.at[core, pl.ds(r0, 125)])


@jax.jit
def _sc_segsum(src3, dst3, table, zd):
    k = pl.kernel(
        _sc_segsum_body,
        out_type=jax.ShapeDtypeStruct((NC, N, D), jnp.float32),
        mesh=_mesh,
        compiler_params=_sc_params,
        scratch_types=[
            pltpu.VMEM_SHARED((N, D), jnp.float32),
            pltpu.VMEM((SNCH, SCH), jnp.int32),
            pltpu.VMEM((SNCH, SCH), jnp.int32),
            pltpu.VMEM((SCH, D), jnp.float32),
            pltpu.VMEM((SCH, D), jnp.float32),
            pltpu.SemaphoreType.DMA,
            pltpu.SemaphoreType.DMA,
            pltpu.SemaphoreType.DMA,
        ],
    )
    return k(src3, dst3, table, zd)


def _sc_gat_body(src3_hbm, dst3_hbm, fs_hbm, fd_hbm, attn_hbm, zd_hbm, z16_hbm,
                 out_hbm, den_hbm,
                 acc_sh, den_sh, is0, id0, is1, id1, a0, a1, b, drows,
                 attn_v, pmat, ebuf, semi0, semi1, sema0, sema1, semb):
    core, sub, base0 = _tile_base()
    wid = core * NS + sub

    ci0s = pltpu.async_copy(src3_hbm.at[wid, 0], is0, semi0)
    ci0d = pltpu.async_copy(dst3_hbm.at[wid, 0], id0, semi0)
    pltpu.sync_copy(zd_hbm.at[pl.ds(sub * RPT, RPT)],
                    acc_sh.at[pl.ds(sub * RPT, RPT)])
    pltpu.sync_copy(z16_hbm.at[pl.ds(sub * RPT, RPT)],
                    den_sh.at[pl.ds(sub * RPT, RPT)])
    pltpu.sync_copy(attn_hbm, attn_v)
    ci0s.wait()
    ci0d.wait()
    pltpu.async_copy(fs_hbm.at[is0], a0, sema0)
    pltpu.async_copy(fd_hbm.at[id0], b, semb)
    pltpu.async_copy(src3_hbm.at[wid, 1], is1, semi1)
    pltpu.async_copy(dst3_hbm.at[wid, 1], id1, semi1)
    plsc.subcore_barrier()

    lanes = lax.iota(jnp.int32, L)
    attn_c = [attn_v[pl.ds(j * L, L)] for j in range(D // L)]

    def lane_bcast(vec, i):
        idx = jnp.full((L, 1), i, jnp.int32)
        dnums = lax.GatherDimensionNumbers(
            offset_dims=(), collapsed_slice_dims=(0,), start_index_map=(0,))
        return lax.gather(vec, idx, dnums, (1,),
                          mode=lax.GatherScatterMode.PROMISE_IN_BOUNDS)

    def logits_phase(arows):
        @pl.loop(0, CH, step=L)
        def _grp(g):
            for i in range(L):
                acc = jnp.zeros((L,), jnp.float32)
                for j in range(D // L):
                    sv = arows[g + i, pl.ds(j * L, L)] + b[g + i, pl.ds(j * L, L)]
                    sv = jnp.where(sv >= 0.0, sv, sv * jnp.float32(0.2))
                    acc = acc + sv * attn_c[j]
                pmat[i, :] = acc
            e16 = jnp.zeros((L,), jnp.float32)
            for kk in range(L):
                e16 = e16 + plsc.load_gather(
                    pmat, [lanes, jnp.full((L,), kk, jnp.int32)])
            ebuf[pl.ds(g, L)] = jnp.exp(e16)

    def scale_phase(arows):
        @pl.loop(0, CH, step=L)
        def _grp(g):
            ee16 = ebuf[pl.ds(g, L)]
            for i in range(L):
                bc = lane_bcast(ee16, i)
                for j in range(D // L):
                    arows[g + i, pl.ds(j * L, L)] = (
                        arows[g + i, pl.ds(j * L, L)] * bc)
                drows[g + i, :] = bc

    def run_chunk(ci, arows, sema, ism, idm, iso, ido, semio, semim, last):
        pltpu.make_async_copy(fs_hbm.at[ism], arows, sema).wait()
        pltpu.make_async_copy(fd_hbm.at[idm], b, semb).wait()
        logits_phase(arows)
        if not last:
            pltpu.make_async_copy(src3_hbm.at[wid, ci + 1], iso, semio).wait()
            pltpu.make_async_copy(dst3_hbm.at[wid, ci + 1], ido, semio).wait()
            other_a = a1 if arows is a0 else a0
            other_sema = sema1 if arows is a0 else sema0
            pltpu.async_copy(fs_hbm.at[iso], other_a, other_sema)
            pltpu.async_copy(fd_hbm.at[ido], b, semb)
        scale_phase(arows)
        pltpu.sync_copy(arows, acc_sh.at[idm], add=True)
        pltpu.sync_copy(drows, den_sh.at[idm], add=True)

    @pl.loop(0, NCHUNK - 1, step=2)
    def _pair(ci0):
        run_chunk(ci0, a0, sema0, is0, id0, is1, id1, semi1, semi0, False)

        @pl.when(ci0 + 2 < NCHUNK)
        def _():
            pltpu.async_copy(src3_hbm.at[wid, ci0 + 2], is0, semi0)
            pltpu.async_copy(dst3_hbm.at[wid, ci0 + 2], id0, semi0)

        run_chunk(ci0 + 1, a1, sema1, is1, id1, is0, id0, semi0, semi1, False)

        @pl.when(ci0 + 3 < NCHUNK)
        def _():
            pltpu.async_copy(src3_hbm.at[wid, ci0 + 3], is1, semi1)
            pltpu.async_copy(dst3_hbm.at[wid, ci0 + 3], id1, semi1)

    run_chunk(NCHUNK - 1, a0, sema0, is0, id0, is1, id1, semi1, semi0, True)

    plsc.subcore_barrier()
    for k in range(RPT // 125):
        r0 = sub * RPT + k * 125
        pltpu.sync_copy(acc_sh.at[pl.ds(r0, 125)], out_hbm.at[core, pl.ds(r0, 125)])
    pltpu.sync_copy(den_sh.at[pl.ds(sub * RPT, RPT)],
                    den_hbm.at[core, pl.ds(sub * RPT, RPT)])


@jax.jit
def _sc_gat(src3, dst3, fs, fd, attn, zd, z16):
    k = pl.kernel(
        _sc_gat_body,
        out_type=(jax.ShapeDtypeStruct((NC, N, D), jnp.float32),
                  jax.ShapeDtypeStruct((NC, N, L), jnp.float32)),
        mesh=_mesh,
        compiler_params=_sc_params,
        scratch_types=[
            pltpu.VMEM_SHARED((N, D), jnp.float32),
            pltpu.VMEM_SHARED((N, L), jnp.float32),
            pltpu.VMEM((CH,), jnp.int32),
            pltpu.VMEM((CH,), jnp.int32),
            pltpu.VMEM((CH,), jnp.int32),
            pltpu.VMEM((CH,), jnp.int32),
            pltpu.VMEM((CH, D), jnp.float32),
            pltpu.VMEM((CH, D), jnp.float32),
            pltpu.VMEM((CH, D), jnp.float32),
            pltpu.VMEM((CH, L), jnp.float32),
            pltpu.VMEM((D,), jnp.float32),
            pltpu.VMEM((L, L), jnp.float32),
            pltpu.VMEM((CH,), jnp.float32),
            pltpu.SemaphoreType.DMA,
            pltpu.SemaphoreType.DMA,
            pltpu.SemaphoreType.DMA,
            pltpu.SemaphoreType.DMA,
            pltpu.SemaphoreType.DMA,
        ],
    )
    return k(src3, dst3, fs, fd, attn, zd, z16)


BLK = 1000
GRID = N // BLK


def _row_spec(width=D):
    return pl.BlockSpec((BLK, width), lambda i: (i, 0))


def _full_spec(shape):
    ndim = len(shape)
    return pl.BlockSpec(shape, lambda i: (0,) * ndim)


def _tc_deg_h_body(d0_ref, d1_ref, x_ref, dinv_ref, h_ref):
    deg = d0_ref[:, 0:1] + d1_ref[:, 0:1]
    dinv = lax.rsqrt(jnp.maximum(deg, 1.0))
    dinv_ref[...] = jnp.broadcast_to(dinv, (BLK, L))
    h_ref[...] = x_ref[...] * dinv


@jax.jit
def _tc_deg_h(degp, x0):
    return pl.pallas_call(
        _tc_deg_h_body,
        grid=(GRID,),
        in_specs=[_row_spec(L), _row_spec(L), _row_spec()],
        out_specs=[_row_spec(L), _row_spec()],
        out_shape=[jax.ShapeDtypeStruct((N, L), jnp.float32),
                   jax.ShapeDtypeStruct((N, D), jnp.float32)],
    )(degp[0], degp[1], x0)


def _tc_step1_body(s0_ref, s1_ref, x0_ref, dinv_ref, rn_ref, x1_ref, h1_ref):
    rn = rn_ref[0]
    dv = dinv_ref[:, 0:1]
    agg = (s0_ref[...] + s1_ref[...]) * dv
    x1 = agg * (-rn) + x0_ref[...] * (rn - 1.0)
    x1_ref[...] = x1
    h1_ref[...] = x1 * dv


@jax.jit
def _tc_step1(s0, s1, x0, dinv, rn):
    return pl.pallas_call(
        _tc_step1_body,
        grid=(GRID,),
        in_specs=[_row_spec(), _row_spec(), _row_spec(), _row_spec(L),
                  pl.BlockSpec(memory_space=pltpu.SMEM)],
        out_specs=[_row_spec(), _row_spec()],
        out_shape=[jax.ShapeDtypeStruct((N, D), jnp.float32),
                   jax.ShapeDtypeStruct((N, D), jnp.float32)],
    )(s0, s1, x0, dinv, rn)


def _tc_step2_body(s0_ref, s1_ref, x0_ref, x1_ref, dinv_ref, rn_ref,
                   w0_ref, w1_ref, w2_ref, b_ref, x_ref, h_ref):
    rn = rn_ref[0]
    dv = dinv_ref[:, 0:1]
    agg = (s0_ref[...] + s1_ref[...]) * dv
    x2 = agg * (-2.0 * rn) + x1_ref[...] * (2.0 * (rn - 1.0)) - x0_ref[...]
    acc = jnp.dot(x0_ref[...], w0_ref[...],
                  preferred_element_type=jnp.float32,
                  precision=lax.Precision.HIGHEST)
    acc += jnp.dot(x1_ref[...], w1_ref[...],
                   preferred_element_type=jnp.float32,
                  precision=lax.Precision.HIGHEST)
    acc += jnp.dot(x2, w2_ref[...], preferred_element_type=jnp.float32,
                  precision=lax.Precision.HIGHEST)
    xn = jnp.maximum(acc + b_ref[...], 0.0)
    x_ref[...] = xn
    h_ref[...] = xn * dv


@jax.jit
def _tc_step2(s0, s1, x0, x1, dinv, rn, w0, w1, w2, b):
    return pl.pallas_call(
        _tc_step2_body,
        grid=(GRID,),
        in_specs=[_row_spec(), _row_spec(), _row_spec(), _row_spec(),
                  _row_spec(L),
                  pl.BlockSpec(memory_space=pltpu.SMEM),
                  _full_spec((D, D)), _full_spec((D, D)), _full_spec((D, D)),
                  _full_spec((1, D))],
        out_specs=[_row_spec(), _row_spec()],
        out_shape=[jax.ShapeDtypeStruct((N, D), jnp.float32),
                   jax.ShapeDtypeStruct((N, D), jnp.float32)],
    )(s0, s1, x0, x1, dinv, rn, w0, w1, w2, b)


def _tc_step2gat_body(s0_ref, s1_ref, x0_ref, x1_ref, dinv_ref, rn_ref,
                      w0_ref, w1_ref, w2_ref, b_ref,
                      ws_ref, bs_ref, wd_ref, bd_ref, fs_ref, fd_ref):
    rn = rn_ref[0]
    dv = dinv_ref[:, 0:1]
    agg = (s0_ref[...] + s1_ref[...]) * dv
    x2 = agg * (-2.0 * rn) + x1_ref[...] * (2.0 * (rn - 1.0)) - x0_ref[...]
    acc = jnp.dot(x0_ref[...], w0_ref[...],
                  preferred_element_type=jnp.float32,
                  precision=lax.Precision.HIGHEST)
    acc += jnp.dot(x1_ref[...], w1_ref[...],
                   preferred_element_type=jnp.float32,
                   precision=lax.Precision.HIGHEST)
    acc += jnp.dot(x2, w2_ref[...], preferred_element_type=jnp.float32,
                   precision=lax.Precision.HIGHEST)
    xn = jnp.maximum(acc + b_ref[...], 0.0)
    fs_ref[...] = jnp.dot(xn, ws_ref[...],
                          preferred_element_type=jnp.float32,
                          precision=lax.Precision.HIGHEST) + bs_ref[...]
    fd_ref[...] = jnp.dot(xn, wd_ref[...],
                          preferred_element_type=jnp.float32,
                          precision=lax.Precision.HIGHEST) + bd_ref[...]


@jax.jit
def _tc_step2gat(s0, s1, x0, x1, dinv, rn, w0, w1, w2, b, ws, bs, wd, bd):
    return pl.pallas_call(
        _tc_step2gat_body,
        grid=(GRID,),
        in_specs=[_row_spec(), _row_spec(), _row_spec(), _row_spec(),
                  _row_spec(L),
                  pl.BlockSpec(memory_space=pltpu.SMEM),
                  _full_spec((D, D)), _full_spec((D, D)), _full_spec((D, D)),
                  _full_spec((1, D)),
                  _full_spec((D, D)), _full_spec((1, D)),
                  _full_spec((D, D)), _full_spec((1, D))],
        out_specs=[_row_spec(), _row_spec()],
        out_shape=[jax.ShapeDtypeStruct((N, D), jnp.float32),
                   jax.ShapeDtypeStruct((N, D), jnp.float32)],
    )(s0, s1, x0, x1, dinv, rn, w0, w1, w2, b, ws, bs, wd, bd)


def _tc_fsfd_body(x_ref, ws_ref, bs_ref, wd_ref, bd_ref, fs_ref, fd_ref):
    x = x_ref[...]
    fs_ref[...] = jnp.dot(x, ws_ref[...],
                          preferred_element_type=jnp.float32,
                  precision=lax.Precision.HIGHEST) + bs_ref[...]
    fd_ref[...] = jnp.dot(x, wd_ref[...],
                          preferred_element_type=jnp.float32,
                  precision=lax.Precision.HIGHEST) + bd_ref[...]


@jax.jit
def _tc_fsfd(x, ws, bs, wd, bd):
    return pl.pallas_call(
        _tc_fsfd_body,
        grid=(GRID,),
        in_specs=[_row_spec(), _full_spec((D, D)), _full_spec((1, D)),
                  _full_spec((D, D)), _full_spec((1, D))],
        out_specs=[_row_spec(), _row_spec()],
        out_shape=[jax.ShapeDtypeStruct((N, D), jnp.float32),
                   jax.ShapeDtypeStruct((N, D), jnp.float32)],
    )(x, ws, bs, wd, bd)


def _tc_final_body(o0_ref, o1_ref, d0_ref, d1_ref, out_ref):
    den = d0_ref[:, 0:1] + d1_ref[:, 0:1]
    num = o0_ref[...] + o1_ref[...]
    out_ref[...] = jnp.where(den > 0.0, num / jnp.maximum(den, 1e-38), 0.0)


@jax.jit
def _tc_final(o0, o1, d0, d1):
    return pl.pallas_call(
        _tc_final_body,
        grid=(GRID,),
        in_specs=[_row_spec(), _row_spec(), _row_spec(L), _row_spec(L)],
        out_specs=_row_spec(),
        out_shape=jax.ShapeDtypeStruct((N, D), jnp.float32),
    )(o0, o1, d0, d1)


def kernel(edge_index, user_embed, laplacian_lambda_max, cheb_W, cheb_b,
           gat_Wsrc, gat_bsrc, gat_Wdst, gat_bdst, gat_attn):
    src = edge_index[0]
    dst = edge_index[1]
    rn = (2.0 / laplacian_lambda_max).astype(jnp.float32)
    w0 = cheb_W[:D]
    w1 = cheb_W[D:2 * D]
    w2 = cheb_W[2 * D:]
    bias = cheb_b.reshape(1, D)
    bs = gat_bsrc.reshape(1, D)
    bd = gat_bdst.reshape(1, D)
    attn = gat_attn.reshape(D)

    zd = jnp.zeros((N, D), jnp.float32)
    z16 = jnp.zeros((N, L), jnp.float32)

    src3 = src.reshape(NW, NCHUNK, CH)
    dst3 = dst.reshape(NW, NCHUNK, CH)
    src3s = src.reshape(NW, SNCH, SCH)
    dst3s = dst.reshape(NW, SNCH, SCH)

    degp = _sc_deg(dst3, z16)
    dinv, h = _tc_deg_h(degp, user_embed)

    def half_cheb(x0, h0):
        s = _sc_segsum(src3s, dst3s, h0, zd)
        x1, h1 = _tc_step1(s[0], s[1], x0, dinv, rn)
        s2 = _sc_segsum(src3s, dst3s, h1, zd)
        return s2, x1

    s2a, x1a = half_cheb(user_embed, h)
    x1, h1 = _tc_step2(s2a[0], s2a[1], user_embed, x1a, dinv, rn,
                       w0, w1, w2, bias)
    s2b, x1b = half_cheb(x1, h1)
    fs, fd = _tc_step2gat(s2b[0], s2b[1], x1, x1b, dinv, rn,
                          w0, w1, w2, bias, gat_Wsrc, bs, gat_Wdst, bd)
    acc, den = _sc_gat(src3, dst3, fs, fd, attn, zd, z16)
    return _tc_final(acc[0], acc[1], den[0], den[1])


# speedup vs baseline: 9.9401x; 1.0008x over previous
import jax
import jax.numpy as jnp
from jax import lax
from jax.experimental import pallas as pl
from jax.experimental.pallas import tpu as pltpu
from jax.experimental.pallas import tpu_sc as plsc

N = 10000
E = 320000
D = 128

NC = 2
NS = 16
NW = NC * NS
L = 16

EPT = E // NW
CH = 80
NCHUNK = EPT // CH
RPT = N // NS
SCH = 100
SNCH = EPT // SCH

_mesh = plsc.VectorSubcoreMesh(core_axis_name="c", subcore_axis_name="s",
                               num_cores=NC, num_subcores=NS)
_sc_params = pltpu.CompilerParams(use_tc_tiling_on_sc=False,
                                  needs_layout_passes=False)


def _tile_base(unused=None):
    core = lax.axis_index("c")
    sub = lax.axis_index("s")
    return core, sub, (core * NS + sub) * EPT


def _sc_deg_body(dst3_hbm, z16_hbm, out_hbm, acc_sh, idxd, ones_v, semi):
    core, sub, base0 = _tile_base()
    wid = core * NS + sub

    cd = pltpu.async_copy(dst3_hbm.at[wid], idxd, semi)
    pltpu.sync_copy(z16_hbm.at[pl.ds(sub * RPT, RPT)],
                    acc_sh.at[pl.ds(sub * RPT, RPT)])

    @pl.loop(0, CH)
    def _ones(i):
        ones_v[i, :] = jnp.ones((L,), jnp.float32)

    cd.wait()
    plsc.subcore_barrier()

    @pl.loop(0, NCHUNK)
    def _chunk(ci):
        pltpu.sync_copy(ones_v, acc_sh.at[idxd.at[ci]], add=True)

    plsc.subcore_barrier()
    pltpu.sync_copy(acc_sh.at[pl.ds(sub * RPT, RPT)],
                    out_hbm.at[core, pl.ds(sub * RPT, RPT)])


@jax.jit
def _sc_deg(dst3, z16):
    k = pl.kernel(
        _sc_deg_body,
        out_type=jax.ShapeDtypeStruct((NC, N, L), jnp.float32),
        mesh=_mesh,
        compiler_params=_sc_params,
        scratch_types=[
            pltpu.VMEM_SHARED((N, L), jnp.float32),
            pltpu.VMEM((NCHUNK, CH), jnp.int32),
            pltpu.VMEM((CH, L), jnp.float32),
            pltpu.SemaphoreType.DMA,
        ],
    )
    return k(dst3, z16)


def _sc_segsum_body(src3_hbm, dst3_hbm, table_hbm, zd_hbm, out_hbm,
                    acc_sh, idxs, idxd, rows0, rows1, semi, sem0, sem1):
    core, sub, base0 = _tile_base()
    wid = core * NS + sub

    cs = pltpu.async_copy(src3_hbm.at[wid], idxs, semi)
    cd = pltpu.async_copy(dst3_hbm.at[wid], idxd, semi)
    pltpu.sync_copy(zd_hbm.at[pl.ds(sub * RPT, RPT)],
                    acc_sh.at[pl.ds(sub * RPT, RPT)])
    cs.wait()
    cd.wait()
    plsc.subcore_barrier()

    pltpu.async_copy(table_hbm.at[idxs.at[0]], rows0, sem0)

    @pl.loop(0, SNCH, step=2)
    def _chunk(ci0):
        pltpu.make_async_copy(table_hbm.at[idxs.at[ci0]], rows0, sem0).wait()
        pltpu.async_copy(table_hbm.at[idxs.at[ci0 + 1]], rows1, sem1)
        pltpu.sync_copy(rows0, acc_sh.at[idxd.at[ci0]], add=True)
        pltpu.make_async_copy(table_hbm.at[idxs.at[ci0 + 1]], rows1, sem1).wait()

        @pl.when(ci0 + 2 < SNCH)
        def _():
            pltpu.async_copy(table_hbm.at[idxs.at[ci0 + 2]], rows0, sem0)

        pltpu.sync_copy(rows1, acc_sh.at[idxd.at[ci0 + 1]], add=True)

    plsc.subcore_barrier()
    for k in range(RPT // 125):
        r0 = sub * RPT + k * 125
        pltpu.sync_copy(acc_sh.at[pl.ds(r0, 125)], out_hbm.at[core, pl.ds(r0, 125)])


@jax.jit
def _sc_segsum(src3, dst3, table, zd):
    k = pl.kernel(
        _sc_segsum_body,
        out_type=jax.ShapeDtypeStruct((NC, N, D), jnp.float32),
        mesh=_mesh,
        compiler_params=_sc_params,
        scratch_types=[
            pltpu.VMEM_SHARED((N, D), jnp.float32),
            pltpu.VMEM((SNCH, SCH), jnp.int32),
            pltpu.VMEM((SNCH, SCH), jnp.int32),
            pltpu.VMEM((SCH, D), jnp.float32),
            pltpu.VMEM((SCH, D), jnp.float32),
            pltpu.SemaphoreType.DMA,
            pltpu.SemaphoreType.DMA,
            pltpu.SemaphoreType.DMA,
        ],
    )
    return k(src3, dst3, table, zd)


def _sc_gat_body(src3_hbm, dst3_hbm, fs_hbm, fd_hbm, attn_hbm, zd_hbm, z16_hbm,
                 out_hbm, den_hbm,
                 acc_sh, den_sh, is0, id0, is1, id1, a0, a1, b, drows,
                 attn_v, pmat, ebuf, semi0, semi1, sema0, sema1, semb):
    core, sub, base0 = _tile_base()
    wid = core * NS + sub

    ci0s = pltpu.async_copy(src3_hbm.at[wid, 0], is0, semi0)
    ci0d = pltpu.async_copy(dst3_hbm.at[wid, 0], id0, semi0)
    pltpu.sync_copy(zd_hbm.at[pl.ds(sub * RPT, RPT)],
                    acc_sh.at[pl.ds(sub * RPT, RPT)])
    pltpu.sync_copy(z16_hbm.at[pl.ds(sub * RPT, RPT)],
                    den_sh.at[pl.ds(sub * RPT, RPT)])
    pltpu.sync_copy(attn_hbm, attn_v)
    ci0s.wait()
    ci0d.wait()
    pltpu.async_copy(fs_hbm.at[is0], a0, sema0)
    pltpu.async_copy(fd_hbm.at[id0], b, semb)
    pltpu.async_copy(src3_hbm.at[wid, 1], is1, semi1)
    pltpu.async_copy(dst3_hbm.at[wid, 1], id1, semi1)
    plsc.subcore_barrier()

    lanes = lax.iota(jnp.int32, L)
    attn_c = [attn_v[pl.ds(j * L, L)] for j in range(D // L)]

    def lane_bcast(vec, i):
        idx = jnp.full((L, 1), i, jnp.int32)
        dnums = lax.GatherDimensionNumbers(
            offset_dims=(), collapsed_slice_dims=(0,), start_index_map=(0,))
        return lax.gather(vec, idx, dnums, (1,),
                          mode=lax.GatherScatterMode.PROMISE_IN_BOUNDS)

    def logits_phase(arows):
        @pl.loop(0, CH, step=L)
        def _grp(g):
            for i in range(L):
                acc = jnp.zeros((L,), jnp.float32)
                for j in range(D // L):
                    sv = arows[g + i, pl.ds(j * L, L)] + b[g + i, pl.ds(j * L, L)]
                    sv = jnp.where(sv >= 0.0, sv, sv * jnp.float32(0.2))
                    acc = acc + sv * attn_c[j]
                pmat[i, :] = acc
            e16 = jnp.zeros((L,), jnp.float32)
            for kk in range(L):
                e16 = e16 + plsc.load_gather(
                    pmat, [lanes, jnp.full((L,), kk, jnp.int32)])
            ebuf[pl.ds(g, L)] = jnp.exp(e16)

    def scale_phase(arows):
        @pl.loop(0, CH, step=L)
        def _grp(g):
            ee16 = ebuf[pl.ds(g, L)]
            for i in range(L):
                bc = lane_bcast(ee16, i)
                for j in range(D // L):
                    arows[g + i, pl.ds(j * L, L)] = (
                        arows[g + i, pl.ds(j * L, L)] * bc)
                drows[g + i, :] = bc

    def run_chunk(ci, arows, sema, ism, idm, iso, ido, semio, semim, last):
        pltpu.make_async_copy(fs_hbm.at[ism], arows, sema).wait()
        pltpu.make_async_copy(fd_hbm.at[idm], b, semb).wait()
        logits_phase(arows)
        if not last:
            pltpu.make_async_copy(src3_hbm.at[wid, ci + 1], iso, semio).wait()
            pltpu.make_async_copy(dst3_hbm.at[wid, ci + 1], ido, semio).wait()
            other_a = a1 if arows is a0 else a0
            other_sema = sema1 if arows is a0 else sema0
            pltpu.async_copy(fs_hbm.at[iso], other_a, other_sema)
            pltpu.async_copy(fd_hbm.at[ido], b, semb)
        scale_phase(arows)
        pltpu.sync_copy(arows, acc_sh.at[idm], add=True)
        pltpu.sync_copy(drows, den_sh.at[idm], add=True)

    @pl.loop(0, NCHUNK - 1, step=2)
    def _pair(ci0):
        run_chunk(ci0, a0, sema0, is0, id0, is1, id1, semi1, semi0, False)

        @pl.when(ci0 + 2 < NCHUNK)
        def _():
            pltpu.async_copy(src3_hbm.at[wid, ci0 + 2], is0, semi0)
            pltpu.async_copy(dst3_hbm.at[wid, ci0 + 2], id0, semi0)

        run_chunk(ci0 + 1, a1, sema1, is1, id1, is0, id0, semi0, semi1, False)

        @pl.when(ci0 + 3 < NCHUNK)
        def _():
            pltpu.async_copy(src3_hbm.at[wid, ci0 + 3], is1, semi1)
            pltpu.async_copy(dst3_hbm.at[wid, ci0 + 3], id1, semi1)

    run_chunk(NCHUNK - 1, a0, sema0, is0, id0, is1, id1, semi1, semi0, True)

    plsc.subcore_barrier()
    for k in range(RPT // 125):
        r0 = sub * RPT + k * 125
        pltpu.sync_copy(acc_sh.at[pl.ds(r0, 125)], out_hbm.at[core, pl.ds(r0, 125)])
    pltpu.sync_copy(den_sh.at[pl.ds(sub * RPT, RPT)],
                    den_hbm.at[core, pl.ds(sub * RPT, RPT)])


@jax.jit
def _sc_gat(src3, dst3, fs, fd, attn, zd, z16):
    k = pl.kernel(
        _sc_gat_body,
        out_type=(jax.ShapeDtypeStruct((NC, N, D), jnp.float32),
                  jax.ShapeDtypeStruct((NC, N, L), jnp.float32)),
        mesh=_mesh,
        compiler_params=_sc_params,
        scratch_types=[
            pltpu.VMEM_SHARED((N, D), jnp.float32),
            pltpu.VMEM_SHARED((N, L), jnp.float32),
            pltpu.VMEM((CH,), jnp.int32),
            pltpu.VMEM((CH,), jnp.int32),
            pltpu.VMEM((CH,), jnp.int32),
            pltpu.VMEM((CH,), jnp.int32),
            pltpu.VMEM((CH, D), jnp.float32),
            pltpu.VMEM((CH, D), jnp.float32),
            pltpu.VMEM((CH, D), jnp.float32),
            pltpu.VMEM((CH, L), jnp.float32),
            pltpu.VMEM((D,), jnp.float32),
            pltpu.VMEM((L, L), jnp.float32),
            pltpu.VMEM((CH,), jnp.float32),
            pltpu.SemaphoreType.DMA,
            pltpu.SemaphoreType.DMA,
            pltpu.SemaphoreType.DMA,
            pltpu.SemaphoreType.DMA,
            pltpu.SemaphoreType.DMA,
        ],
    )
    return k(src3, dst3, fs, fd, attn, zd, z16)


BLK = 1000
GRID = N // BLK


def _row_spec(width=D):
    return pl.BlockSpec((BLK, width), lambda i: (i, 0))


def _full_spec(shape):
    ndim = len(shape)
    return pl.BlockSpec(shape, lambda i: (0,) * ndim)


def _tc_deg_h_body(d0_ref, d1_ref, x_ref, dinv_ref, h_ref):
    deg = d0_ref[:, 0:1] + d1_ref[:, 0:1]
    dinv = lax.rsqrt(jnp.maximum(deg, 1.0))
    dinv_ref[...] = jnp.broadcast_to(dinv, (BLK, L))
    h_ref[...] = x_ref[...] * dinv


@jax.jit
def _tc_deg_h(degp, x0):
    return pl.pallas_call(
        _tc_deg_h_body,
        grid=(GRID,),
        in_specs=[_row_spec(L), _row_spec(L), _row_spec()],
        out_specs=[_row_spec(L), _row_spec()],
        out_shape=[jax.ShapeDtypeStruct((N, L), jnp.float32),
                   jax.ShapeDtypeStruct((N, D), jnp.float32)],
    )(degp[0], degp[1], x0)


def _tc_step1_body(s0_ref, s1_ref, x0_ref, dinv_ref, rn_ref, x1_ref, h1_ref):
    rn = rn_ref[0]
    dv = dinv_ref[:, 0:1]
    agg = (s0_ref[...] + s1_ref[...]) * dv
    x1 = agg * (-rn) + x0_ref[...] * (rn - 1.0)
    x1_ref[...] = x1
    h1_ref[...] = x1 * dv


@jax.jit
def _tc_step1(s0, s1, x0, dinv, rn):
    return pl.pallas_call(
        _tc_step1_body,
        grid=(GRID,),
        in_specs=[_row_spec(), _row_spec(), _row_spec(), _row_spec(L),
                  pl.BlockSpec(memory_space=pltpu.SMEM)],
        out_specs=[_row_spec(), _row_spec()],
        out_shape=[jax.ShapeDtypeStruct((N, D), jnp.float32),
                   jax.ShapeDtypeStruct((N, D), jnp.float32)],
    )(s0, s1, x0, dinv, rn)


def _tc_step2_body(s0_ref, s1_ref, x0_ref, x1_ref, dinv_ref, rn_ref,
                   w0_ref, w1_ref, w2_ref, b_ref, x_ref, h_ref):
    rn = rn_ref[0]
    dv = dinv_ref[:, 0:1]
    agg = (s0_ref[...] + s1_ref[...]) * dv
    x2 = agg * (-2.0 * rn) + x1_ref[...] * (2.0 * (rn - 1.0)) - x0_ref[...]
    acc = jnp.dot(x0_ref[...], w0_ref[...],
                  preferred_element_type=jnp.float32,
                  precision=lax.Precision.HIGHEST)
    acc += jnp.dot(x1_ref[...], w1_ref[...],
                   preferred_element_type=jnp.float32,
                  precision=lax.Precision.HIGHEST)
    acc += jnp.dot(x2, w2_ref[...], preferred_element_type=jnp.float32,
                  precision=lax.Precision.HIGHEST)
    xn = jnp.maximum(acc + b_ref[...], 0.0)
    x_ref[...] = xn
    h_ref[...] = xn * dv


@jax.jit
def _tc_step2(s0, s1, x0, x1, dinv, rn, w0, w1, w2, b):
    return pl.pallas_call(
        _tc_step2_body,
        grid=(GRID,),
        in_specs=[_row_spec(), _row_spec(), _row_spec(), _row_spec(),
                  _row_spec(L),
                  pl.BlockSpec(memory_space=pltpu.SMEM),
                  _full_spec((D, D)), _full_spec((D, D)), _full_spec((D, D)),
                  _full_spec((1, D))],
        out_specs=[_row_spec(), _row_spec()],
        out_shape=[jax.ShapeDtypeStruct((N, D), jnp.float32),
                   jax.ShapeDtypeStruct((N, D), jnp.float32)],
    )(s0, s1, x0, x1, dinv, rn, w0, w1, w2, b)


def _tc_step2gat_body(s0_ref, s1_ref, x0_ref, x1_ref, dinv_ref, rn_ref,
                      w0_ref, w1_ref, w2_ref, b_ref,
                      ws_ref, bs_ref, wd_ref, bd_ref, fs_ref, fd_ref):
    rn = rn_ref[0]
    dv = dinv_ref[:, 0:1]
    agg = (s0_ref[...] + s1_ref[...]) * dv
    x2 = agg * (-2.0 * rn) + x1_ref[...] * (2.0 * (rn - 1.0)) - x0_ref[...]
    acc = jnp.dot(x0_ref[...], w0_ref[...],
                  preferred_element_type=jnp.float32,
                  precision=lax.Precision.HIGHEST)
    acc += jnp.dot(x1_ref[...], w1_ref[...],
                   preferred_element_type=jnp.float32,
                   precision=lax.Precision.HIGHEST)
    acc += jnp.dot(x2, w2_ref[...], preferred_element_type=jnp.float32,
                   precision=lax.Precision.HIGHEST)
    xn = jnp.maximum(acc + b_ref[...], 0.0)
    fs_ref[...] = jnp.dot(xn, ws_ref[...],
                          preferred_element_type=jnp.float32,
                          precision=lax.Precision.HIGHEST) + bs_ref[...]
    fd_ref[...] = jnp.dot(xn, wd_ref[...],
                          preferred_element_type=jnp.float32,
                          precision=lax.Precision.HIGHEST) + bd_ref[...]


@jax.jit
def _tc_step2gat(s0, s1, x0, x1, dinv, rn, w0, w1, w2, b, ws, bs, wd, bd):
    return pl.pallas_call(
        _tc_step2gat_body,
        grid=(GRID,),
        in_specs=[_row_spec(), _row_spec(), _row_spec(), _row_spec(),
                  _row_spec(L),
                  pl.BlockSpec(memory_space=pltpu.SMEM),
                  _full_spec((D, D)), _full_spec((D, D)), _full_spec((D, D)),
                  _full_spec((1, D)),
                  _full_spec((D, D)), _full_spec((1, D)),
                  _full_spec((D, D)), _full_spec((1, D))],
        out_specs=[_row_spec(), _row_spec()],
        out_shape=[jax.ShapeDtypeStruct((N, D), jnp.float32),
                   jax.ShapeDtypeStruct((N, D), jnp.float32)],
    )(s0, s1, x0, x1, dinv, rn, w0, w1, w2, b, ws, bs, wd, bd)


def _tc_final_body(o0_ref, o1_ref, d0_ref, d1_ref, out_ref):
    den = d0_ref[:, 0:1] + d1_ref[:, 0:1]
    num = o0_ref[...] + o1_ref[...]
    out_ref[...] = jnp.where(den > 0.0, num / jnp.maximum(den, 1e-38), 0.0)


@jax.jit
def _tc_final(o0, o1, d0, d1):
    return pl.pallas_call(
        _tc_final_body,
        grid=(GRID,),
        in_specs=[_row_spec(), _row_spec(), _row_spec(L), _row_spec(L)],
        out_specs=_row_spec(),
        out_shape=jax.ShapeDtypeStruct((N, D), jnp.float32),
    )(o0, o1, d0, d1)


def kernel(edge_index, user_embed, laplacian_lambda_max, cheb_W, cheb_b,
           gat_Wsrc, gat_bsrc, gat_Wdst, gat_bdst, gat_attn):
    src = edge_index[0]
    dst = edge_index[1]
    rn = (2.0 / laplacian_lambda_max).astype(jnp.float32)
    w0 = cheb_W[:D]
    w1 = cheb_W[D:2 * D]
    w2 = cheb_W[2 * D:]
    bias = cheb_b.reshape(1, D)
    bs = gat_bsrc.reshape(1, D)
    bd = gat_bdst.reshape(1, D)
    attn = gat_attn.reshape(D)

    zd = jnp.zeros((N, D), jnp.float32)
    z16 = jnp.zeros((N, L), jnp.float32)

    src3 = src.reshape(NW, NCHUNK, CH)
    dst3 = dst.reshape(NW, NCHUNK, CH)
    src3s = src.reshape(NW, SNCH, SCH)
    dst3s = dst.reshape(NW, SNCH, SCH)

    degp = _sc_deg(dst3, z16)
    dinv, h = _tc_deg_h(degp, user_embed)

    def half_cheb(x0, h0):
        s = _sc_segsum(src3s, dst3s, h0, zd)
        x1, h1 = _tc_step1(s[0], s[1], x0, dinv, rn)
        s2 = _sc_segsum(src3s, dst3s, h1, zd)
        return s2, x1

    s2a, x1a = half_cheb(user_embed, h)
    x1, h1 = _tc_step2(s2a[0], s2a[1], user_embed, x1a, dinv, rn,
                       w0, w1, w2, bias)
    s2b, x1b = half_cheb(x1, h1)
    fs, fd = _tc_step2gat(s2b[0], s2b[1], x1, x1b, dinv, rn,
                          w0, w1, w2, bias, gat_Wsrc, bs, gat_Wdst, bd)
    acc, den = _sc_gat(src3, dst3, fs, fd, attn, zd, z16)
    return _tc_final(acc[0], acc[1], den[0], den[1])
